# Initial kernel scaffold; baseline (speedup 1.0000x reference)
import jax
import jax.numpy as jnp
from jax.experimental import pallas as pl


def kernel(sampled_probs, parent_probs, sample_k, sample_min_prob):
    raise NotImplementedError("write your pallas kernel here")


# speedup vs baseline: 17.9924x; 17.9924x over previous
import functools

import jax
import jax.numpy as jnp
from jax import lax
from jax.experimental import pallas as pl
from jax.experimental.pallas import tpu as pltpu
from jax.experimental.pallas import tpu_sc as plsc

B, N, V = 32, 16, 100000
K = 64
GRP = 800
GPS = V // GRP
G = N * GPS
NGV = G // 16
GV = GRP // 16
ROWS_BLK = 640

_NEG = -1.0
_BIG = 0x7FFFFFFF


def _groupmax_body(x_ref, p_ref, o_ref):
    o_ref[...] = jnp.max(x_ref[...], axis=1, keepdims=True) * p_ref[...]


def _phase_a(raw2d, par_exp):
    rows = raw2d.shape[0]
    return pl.pallas_call(
        _groupmax_body,
        grid=(rows // ROWS_BLK,),
        in_specs=[
            pl.BlockSpec((ROWS_BLK, GRP), lambda i: (i, 0)),
            pl.BlockSpec((ROWS_BLK, 1), lambda i: (i, 0)),
        ],
        out_specs=pl.BlockSpec((ROWS_BLK, 1), lambda i: (i, 0)),
        out_shape=jax.ShapeDtypeStruct((rows, 1), jnp.float32),
    )(raw2d, par_exp)


def _max16(v):
    m = v[0]
    for l in range(1, 16):
        m = jnp.maximum(m, v[l])
    return m


def _argmax16_base(v, base):
    m, mi = v[0], base
    for l in range(1, 16):
        better = v[l] > m
        m = jnp.where(better, v[l], m)
        mi = jnp.where(better, base + l, mi)
    return m, mi


def _argmax16_pair(mv, iv):
    m, mi = mv[0], iv[0]
    for l in range(1, 16):
        better = (mv[l] > m) | ((mv[l] == m) & (iv[l] < mi))
        m = jnp.where(better, mv[l], m)
        mi = jnp.where(better, iv[l], mi)
    return m, mi


def _argmax16_triple(mv, iv, jv):
    m, mi, mj = mv[0], iv[0], jv[0]
    for l in range(1, 16):
        better = (mv[l] > m) | ((mv[l] == m) & (iv[l] < mi))
        m = jnp.where(better, mv[l], m)
        mi = jnp.where(better, iv[l], mi)
        mj = jnp.where(better, jv[l], mj)
    return m, mi, mj


def _select16(v, sel):
    acc = v[0]
    for l in range(1, 16):
        acc = jnp.where(sel == l, v[l], acc)
    return acc


def _rmw_store(ref, lanes, slot, val):
    off = (slot // 16) * 16
    vec = ref[pl.ds(off, 16)]
    ref[pl.ds(off, 16)] = jnp.where(lanes == slot - off, val, vec)


def _phase_b_body(sgm_hbm, par_hbm, raw_hbm,
                  tok_hbm, prb_hbm, pidx_hbm,
                  gm_v, par_v, pvm_v, selgid_v, grp_v,
                  gbv_v, gbi_v, tok_v, prb_v, pidx_v, sem):
    b = lax.axis_index("s") * 2 + lax.axis_index("c")
    lanes = lax.iota(jnp.int32, 16)

    pltpu.sync_copy(sgm_hbm.at[b], gm_v)
    pltpu.sync_copy(par_hbm.at[b], par_v)
    pvec = par_v[pl.ds(0, 16)]
    par_s = [pvec[l] for l in range(16)]

    pvm_v[pl.ds(0, 16)] = jnp.full((16,), _NEG, jnp.float32)
    for k in range(1, 8):
        pvm_v[pl.ds(k * 16, 16)] = jnp.full((16,), _NEG, jnp.float32)

    def vreg_max(i, c):
        _rmw_store(pvm_v, lanes, i, _max16(gm_v[pl.ds(i * 16, 16)]))
        return c
    lax.fori_loop(0, NGV, vreg_max, 0)

    def pick_group(t, c):
        mv = jnp.full((16,), -2.0, jnp.float32)
        sv = jnp.full((16,), _BIG, jnp.int32)
        for k in range(8):
            v = pvm_v[pl.ds(k * 16, 16)]
            take = v > mv
            mv = jnp.where(take, v, mv)
            sv = jnp.where(take, k * 16 + lanes, sv)
        _, s_win = _argmax16_pair(mv, sv)
        vwin = gm_v[pl.ds(s_win * 16, 16)]
        _, gid = _argmax16_base(vwin, s_win * 16)
        _rmw_store(selgid_v, lanes, t, gid)
        vnew = jnp.where(lanes == gid - s_win * 16, jnp.float32(_NEG), vwin)
        gm_v[pl.ds(s_win * 16, 16)] = vnew
        _rmw_store(pvm_v, lanes, s_win, _max16(vnew))
        return c
    lax.fori_loop(0, K, pick_group, 0)

    row_base = b * (N * V)
    for vi in range(4):
        gvec = selgid_v[pl.ds(vi * 16, 16)]
        for l in range(16):
            j = vi * 16 + l
            src = row_base + gvec[l] * GRP
            pltpu.async_copy(raw_hbm.at[pl.ds(src, GRP)],
                             grp_v.at[pl.ds(j * GRP, GRP)], sem)

    def drain(j, c):
        pltpu.make_async_copy(raw_hbm.at[pl.ds(0, GRP)],
                              grp_v.at[pl.ds(j * GRP, GRP)], sem).wait()
        return c
    lax.fori_loop(0, K, drain, 0)

    def group_init(j, c):
        gvec = selgid_v[pl.ds((j // 16) * 16, 16)]
        gid = _select16(gvec, j - (j // 16) * 16)
        scale = _select16(par_s, gid // GPS)
        fbase = gid * GRP

        def fold(v, carry):
            mv, iv = carry
            off = j * GRP + v * 16
            val = grp_v[pl.ds(off, 16)] * scale
            grp_v[pl.ds(off, 16)] = val
            fids = fbase + v * 16 + lanes
            take = (val > mv) | ((val == mv) & (fids < iv))
            return jnp.where(take, val, mv), jnp.where(take, fids, iv)

        mv, iv = lax.fori_loop(
            0, GV, fold,
            (jnp.full((16,), -2.0, jnp.float32),
             jnp.full((16,), _BIG, jnp.int32)))
        bv, bi = _argmax16_pair(mv, iv)
        _rmw_store(gbv_v, lanes, j, bv)
        _rmw_store(gbi_v, lanes, j, bi)
        return c
    lax.fori_loop(0, K, group_init, 0)

    def round_t(t, c):
        mv = jnp.full((16,), -2.0, jnp.float32)
        iv = jnp.full((16,), _BIG, jnp.int32)
        jv = jnp.full((16,), _BIG, jnp.int32)
        for k in range(4):
            v = gbv_v[pl.ds(k * 16, 16)]
            fi = gbi_v[pl.ds(k * 16, 16)]
            take = (v > mv) | ((v == mv) & (fi < iv))
            mv = jnp.where(take, v, mv)
            iv = jnp.where(take, fi, iv)
            jv = jnp.where(take, k * 16 + lanes, jv)
        m, fwin, jwin = _argmax16_triple(mv, iv, jv)

        _rmw_store(prb_v, lanes, t, m)
        _rmw_store(tok_v, lanes, t, fwin % V)
        _rmw_store(pidx_v, lanes, t, fwin // V)

        gid = fwin // GRP
        local = fwin - gid * GRP
        voff = jwin * GRP + (local // 16) * 16
        vec = grp_v[pl.ds(voff, 16)]
        grp_v[pl.ds(voff, 16)] = jnp.where(lanes == local - (local // 16) * 16,
                                           jnp.float32(_NEG), vec)

        fbase = gid * GRP
        jbase = jwin * GRP

        def fold(v, carry):
            mv2, iv2 = carry
            val = grp_v[pl.ds(jbase + v * 16, 16)]
            fids = fbase + v * 16 + lanes
            take = (val > mv2) | ((val == mv2) & (fids < iv2))
            return jnp.where(take, val, mv2), jnp.where(take, fids, iv2)

        mv2, iv2 = lax.fori_loop(
            0, GV, fold,
            (jnp.full((16,), -2.0, jnp.float32),
             jnp.full((16,), _BIG, jnp.int32)))
        bv, bi = _argmax16_pair(mv2, iv2)
        _rmw_store(gbv_v, lanes, jwin, bv)
        _rmw_store(gbi_v, lanes, jwin, bi)
        return c
    lax.fori_loop(0, K, round_t, 0)

    pltpu.sync_copy(tok_v, tok_hbm.at[b])
    pltpu.sync_copy(prb_v, prb_hbm.at[b])
    pltpu.sync_copy(pidx_v, pidx_hbm.at[b])


def _phase_b(sgm, parent_probs, raw1d):
    mesh = plsc.VectorSubcoreMesh(core_axis_name="c", subcore_axis_name="s")
    fn = functools.partial(
        pl.kernel,
        mesh=mesh,
        out_type=[
            jax.ShapeDtypeStruct((B, K), jnp.int32),
            jax.ShapeDtypeStruct((B, K), jnp.float32),
            jax.ShapeDtypeStruct((B, K), jnp.int32),
        ],
        scratch_types=[
            pltpu.VMEM((G,), jnp.float32),
            pltpu.VMEM((16,), jnp.float32),
            pltpu.VMEM((128,), jnp.float32),
            pltpu.VMEM((K,), jnp.int32),
            pltpu.VMEM((K * GRP,), jnp.float32),
            pltpu.VMEM((K,), jnp.float32),
            pltpu.VMEM((K,), jnp.int32),
            pltpu.VMEM((K,), jnp.int32),
            pltpu.VMEM((K,), jnp.float32),
            pltpu.VMEM((K,), jnp.int32),
            pltpu.SemaphoreType.DMA,
        ],
    )(_phase_b_body)
    return fn(sgm, parent_probs, raw1d)


def kernel(sampled_probs, parent_probs, sample_k, sample_min_prob):
    del sample_k, sample_min_prob
    raw2d = sampled_probs.reshape(B * G, GRP)
    par_exp = jnp.repeat(parent_probs.reshape(-1), GPS)[:, None]
    sgm = _phase_a(raw2d, par_exp).reshape(B, G)
    raw1d = sampled_probs.reshape(B * N * V)
    tok, prb, pidx = _phase_b(sgm, parent_probs, raw1d)
    return tok, prb, pidx


# speedup vs baseline: 26.5206x; 1.4740x over previous
import functools

import jax
import jax.numpy as jnp
from jax import lax
from jax.experimental import pallas as pl
from jax.experimental.pallas import tpu as pltpu
from jax.experimental.pallas import tpu_sc as plsc

B, N, V = 32, 16, 100000
K = 64
GRP = 800
GPS = V // GRP
G = N * GPS
NGV = G // 16
GV = GRP // 16
ROWS_BLK = 640

_NEG = -1.0
_BIG = 0x7FFFFFFF


A_ROWS = 8


def _groupmax_body(x_ref, p_ref, o_ref):
    x = x_ref[...]
    m = jnp.max(x.reshape(A_ROWS, GPS, GRP), axis=2)
    o_ref[...] = m * p_ref[...]


def _phase_a(raw2d, par2d):
    return pl.pallas_call(
        _groupmax_body,
        grid=(B * N // A_ROWS,),
        in_specs=[
            pl.BlockSpec((A_ROWS, V), lambda i: (i, 0)),
            pl.BlockSpec((A_ROWS, 1), lambda i: (i, 0)),
        ],
        out_specs=pl.BlockSpec((A_ROWS, GPS), lambda i: (i, 0)),
        out_shape=jax.ShapeDtypeStruct((B * N, GPS), jnp.float32),
    )(raw2d, par2d)


def _max16(v):
    m = v[0]
    for l in range(1, 16):
        m = jnp.maximum(m, v[l])
    return m


def _argmax16_base(v, base):
    m, mi = v[0], base
    for l in range(1, 16):
        better = v[l] > m
        m = jnp.where(better, v[l], m)
        mi = jnp.where(better, base + l, mi)
    return m, mi


def _argmax16_pair(mv, iv):
    m, mi = mv[0], iv[0]
    for l in range(1, 16):
        better = (mv[l] > m) | ((mv[l] == m) & (iv[l] < mi))
        m = jnp.where(better, mv[l], m)
        mi = jnp.where(better, iv[l], mi)
    return m, mi


def _argmax16_triple(mv, iv, jv):
    m, mi, mj = mv[0], iv[0], jv[0]
    for l in range(1, 16):
        better = (mv[l] > m) | ((mv[l] == m) & (iv[l] < mi))
        m = jnp.where(better, mv[l], m)
        mi = jnp.where(better, iv[l], mi)
        mj = jnp.where(better, jv[l], mj)
    return m, mi, mj


def _select16(v, sel):
    acc = v[0]
    for l in range(1, 16):
        acc = jnp.where(sel == l, v[l], acc)
    return acc


def _rmw_store(ref, lanes, slot, val):
    off = (slot // 16) * 16
    vec = ref[pl.ds(off, 16)]
    ref[pl.ds(off, 16)] = jnp.where(lanes == slot - off, val, vec)


def _phase_b_body(sgm_hbm, par_hbm, raw_hbm,
                  tok_hbm, prb_hbm, pidx_hbm,
                  gm_v, par_v, pvm_v, selgid_v, grp_v,
                  gbv_v, gbi_v, tok_v, prb_v, pidx_v, sem):
    b = lax.axis_index("s") * 2 + lax.axis_index("c")
    lanes = lax.iota(jnp.int32, 16)

    pltpu.sync_copy(sgm_hbm.at[b], gm_v)
    pltpu.sync_copy(par_hbm.at[b], par_v)
    pvec = par_v[pl.ds(0, 16)]
    par_s = [pvec[l] for l in range(16)]

    pvm_v[pl.ds(0, 16)] = jnp.full((16,), _NEG, jnp.float32)
    for k in range(1, 8):
        pvm_v[pl.ds(k * 16, 16)] = jnp.full((16,), _NEG, jnp.float32)

    def vreg_max(i, c):
        _rmw_store(pvm_v, lanes, i, _max16(gm_v[pl.ds(i * 16, 16)]))
        return c
    lax.fori_loop(0, NGV, vreg_max, 0)

    def pick_group(t, c):
        mv = jnp.full((16,), -2.0, jnp.float32)
        sv = jnp.full((16,), _BIG, jnp.int32)
        for k in range(8):
            v = pvm_v[pl.ds(k * 16, 16)]
            take = v > mv
            mv = jnp.where(take, v, mv)
            sv = jnp.where(take, k * 16 + lanes, sv)
        _, s_win = _argmax16_pair(mv, sv)
        vwin = gm_v[pl.ds(s_win * 16, 16)]
        _, gid = _argmax16_base(vwin, s_win * 16)
        _rmw_store(selgid_v, lanes, t, gid)
        vnew = jnp.where(lanes == gid - s_win * 16, jnp.float32(_NEG), vwin)
        gm_v[pl.ds(s_win * 16, 16)] = vnew
        _rmw_store(pvm_v, lanes, s_win, _max16(vnew))
        return c
    lax.fori_loop(0, K, pick_group, 0)

    row_base = b * (N * V)
    for vi in range(4):
        gvec = selgid_v[pl.ds(vi * 16, 16)]
        for l in range(16):
            j = vi * 16 + l
            src = row_base + gvec[l] * GRP
            pltpu.async_copy(raw_hbm.at[pl.ds(src, GRP)],
                             grp_v.at[pl.ds(j * GRP, GRP)], sem)

    def drain(j, c):
        pltpu.make_async_copy(raw_hbm.at[pl.ds(0, GRP)],
                              grp_v.at[pl.ds(j * GRP, GRP)], sem).wait()
        return c
    lax.fori_loop(0, K, drain, 0)

    def group_init(j, c):
        gvec = selgid_v[pl.ds((j // 16) * 16, 16)]
        gid = _select16(gvec, j - (j // 16) * 16)
        scale = _select16(par_s, gid // GPS)
        fbase = gid * GRP

        def fold(v, carry):
            mv, iv = carry
            off = j * GRP + v * 16
            val = grp_v[pl.ds(off, 16)] * scale
            grp_v[pl.ds(off, 16)] = val
            fids = fbase + v * 16 + lanes
            take = (val > mv) | ((val == mv) & (fids < iv))
            return jnp.where(take, val, mv), jnp.where(take, fids, iv)

        mv, iv = lax.fori_loop(
            0, GV, fold,
            (jnp.full((16,), -2.0, jnp.float32),
             jnp.full((16,), _BIG, jnp.int32)))
        bv, bi = _argmax16_pair(mv, iv)
        _rmw_store(gbv_v, lanes, j, bv)
        _rmw_store(gbi_v, lanes, j, bi)
        return c
    lax.fori_loop(0, K, group_init, 0)

    def round_t(t, c):
        mv = jnp.full((16,), -2.0, jnp.float32)
        iv = jnp.full((16,), _BIG, jnp.int32)
        jv = jnp.full((16,), _BIG, jnp.int32)
        for k in range(4):
            v = gbv_v[pl.ds(k * 16, 16)]
            fi = gbi_v[pl.ds(k * 16, 16)]
            take = (v > mv) | ((v == mv) & (fi < iv))
            mv = jnp.where(take, v, mv)
            iv = jnp.where(take, fi, iv)
            jv = jnp.where(take, k * 16 + lanes, jv)
        m, fwin, jwin = _argmax16_triple(mv, iv, jv)

        _rmw_store(prb_v, lanes, t, m)
        _rmw_store(tok_v, lanes, t, fwin % V)
        _rmw_store(pidx_v, lanes, t, fwin // V)

        gid = fwin // GRP
        local = fwin - gid * GRP
        voff = jwin * GRP + (local // 16) * 16
        vec = grp_v[pl.ds(voff, 16)]
        grp_v[pl.ds(voff, 16)] = jnp.where(lanes == local - (local // 16) * 16,
                                           jnp.float32(_NEG), vec)

        fbase = gid * GRP
        jbase = jwin * GRP

        def fold(v, carry):
            mv2, iv2 = carry
            val = grp_v[pl.ds(jbase + v * 16, 16)]
            fids = fbase + v * 16 + lanes
            take = (val > mv2) | ((val == mv2) & (fids < iv2))
            return jnp.where(take, val, mv2), jnp.where(take, fids, iv2)

        mv2, iv2 = lax.fori_loop(
            0, GV, fold,
            (jnp.full((16,), -2.0, jnp.float32),
             jnp.full((16,), _BIG, jnp.int32)))
        bv, bi = _argmax16_pair(mv2, iv2)
        _rmw_store(gbv_v, lanes, jwin, bv)
        _rmw_store(gbi_v, lanes, jwin, bi)
        return c
    lax.fori_loop(0, K, round_t, 0)

    pltpu.sync_copy(tok_v, tok_hbm.at[b])
    pltpu.sync_copy(prb_v, prb_hbm.at[b])
    pltpu.sync_copy(pidx_v, pidx_hbm.at[b])


def _phase_b(sgm, parent_probs, raw1d):
    mesh = plsc.VectorSubcoreMesh(core_axis_name="c", subcore_axis_name="s")
    fn = functools.partial(
        pl.kernel,
        mesh=mesh,
        out_type=[
            jax.ShapeDtypeStruct((B, K), jnp.int32),
            jax.ShapeDtypeStruct((B, K), jnp.float32),
            jax.ShapeDtypeStruct((B, K), jnp.int32),
        ],
        scratch_types=[
            pltpu.VMEM((G,), jnp.float32),
            pltpu.VMEM((16,), jnp.float32),
            pltpu.VMEM((128,), jnp.float32),
            pltpu.VMEM((K,), jnp.int32),
            pltpu.VMEM((K * GRP,), jnp.float32),
            pltpu.VMEM((K,), jnp.float32),
            pltpu.VMEM((K,), jnp.int32),
            pltpu.VMEM((K,), jnp.int32),
            pltpu.VMEM((K,), jnp.float32),
            pltpu.VMEM((K,), jnp.int32),
            pltpu.SemaphoreType.DMA,
        ],
    )(_phase_b_body)
    return fn(sgm, parent_probs, raw1d)


def kernel(sampled_probs, parent_probs, sample_k, sample_min_prob):
    del sample_k, sample_min_prob
    flat = sampled_probs.reshape(B * N * V)
    raw2d = sampled_probs.reshape(B * N, V)
    par2d = parent_probs.reshape(B * N, 1)
    sgm = _phase_a(raw2d, par2d).reshape(B, G)
    tok, prb, pidx = _phase_b(sgm, parent_probs, flat)
    return tok, prb, pidx


# speedup vs baseline: 64.6838x; 2.4390x over previous
import functools

import jax
import jax.numpy as jnp
from jax import lax
from jax.experimental import pallas as pl
from jax.experimental.pallas import tpu as pltpu
from jax.experimental.pallas import tpu_sc as plsc

B, N, V = 32, 16, 100000
K = 64
SEL = 72
NSLOT = SEL + 2
TILES = V // 128
TAIL0 = TILES * 128
GPB = TILES + 1
SLOT_PITCH = 800
GMLEN = 2 * SLOT_PITCH
NGV = GMLEN // 16
NPV = 7
A_ROWS = 8

_NEG = -1.0
_BIG = 0x7FFFFFFF


def _groupmax_body(x_ref, p_ref, o_ref):
    xs = x_ref[...] * p_ref[...]
    main = jnp.max(xs[:, :TAIL0].reshape(A_ROWS, TILES, 128), axis=(0, 2))
    tail = jnp.max(xs[:, TAIL0:])
    pad = jnp.full((SLOT_PITCH - GPB,), _NEG, jnp.float32)
    o_ref[...] = jnp.concatenate(
        [main, tail.reshape(1), pad]).reshape(1, 1, SLOT_PITCH)


def _phase_a(raw2d, par2d):
    return pl.pallas_call(
        _groupmax_body,
        grid=(B * N // A_ROWS,),
        in_specs=[
            pl.BlockSpec((A_ROWS, V), lambda i: (i, 0)),
            pl.BlockSpec((A_ROWS, 1), lambda i: (i, 0)),
        ],
        out_specs=pl.BlockSpec((1, 1, SLOT_PITCH), lambda i: (i, 0, 0)),
        out_shape=jax.ShapeDtypeStruct(
            (B * N // A_ROWS, 1, SLOT_PITCH), jnp.float32),
    )(raw2d, par2d)


def _max16(v):
    m = v[0]
    for l in range(1, 16):
        m = jnp.maximum(m, v[l])
    return m


def _argmax16_base(v, base):
    m, mi = v[0], base
    for l in range(1, 16):
        better = v[l] > m
        m = jnp.where(better, v[l], m)
        mi = jnp.where(better, base + l, mi)
    return m, mi


def _argmax16_pair(mv, iv):
    m, mi = mv[0], iv[0]
    for l in range(1, 16):
        better = (mv[l] > m) | ((mv[l] == m) & (iv[l] < mi))
        m = jnp.where(better, mv[l], m)
        mi = jnp.where(better, iv[l], mi)
    return m, mi


def _argmax16_triple(mv, iv, jv):
    m, mi, mj = mv[0], iv[0], jv[0]
    for l in range(1, 16):
        better = (mv[l] > m) | ((mv[l] == m) & (iv[l] < mi))
        m = jnp.where(better, mv[l], m)
        mi = jnp.where(better, iv[l], mi)
        mj = jnp.where(better, jv[l], mj)
    return m, mi, mj


def _select16(vals, sel):
    acc = vals[0]
    for l in range(1, 16):
        acc = jnp.where(sel == l, vals[l], acc)
    return acc


def _rmw_store(ref, lanes, slot, val):
    off = (slot // 16) * 16
    vec = ref[pl.ds(off, 16)]
    ref[pl.ds(off, 16)] = jnp.where(lanes == slot - off, val, vec)


def _phase_b_body(sgm_hbm, par_hbm, raw_hbm,
                  tok_hbm, prb_hbm, pidx_hbm,
                  gm_v, par_v, pvm_v, selgid_v, fb_v, grp_v, tail_v,
                  gbv_v, gbi_v, tok_v, prb_v, pidx_v, sem):
    b = lax.axis_index("s") * 2 + lax.axis_index("c")
    lanes = lax.iota(jnp.int32, 16)

    pltpu.sync_copy(sgm_hbm.at[pl.ds(b * GMLEN, GMLEN)], gm_v)
    pltpu.sync_copy(par_hbm.at[b], par_v)
    pvec = par_v[pl.ds(0, 16)]
    par_s = [pvec[l] for l in range(16)]

    for k in range(NPV):
        pvm_v[pl.ds(k * 16, 16)] = jnp.full((16,), _NEG, jnp.float32)

    def vreg_max(i, c):
        _rmw_store(pvm_v, lanes, i, _max16(gm_v[pl.ds(i * 16, 16)]))
        return c
    lax.fori_loop(0, NGV, vreg_max, 0)

    def pick_group(t, c):
        mv = jnp.full((16,), -2.0, jnp.float32)
        sv = jnp.full((16,), _BIG, jnp.int32)
        for k in range(NPV):
            v = pvm_v[pl.ds(k * 16, 16)]
            take = v > mv
            mv = jnp.where(take, v, mv)
            sv = jnp.where(take, k * 16 + lanes, sv)
        _, s_win = _argmax16_pair(mv, sv)
        vwin = gm_v[pl.ds(s_win * 16, 16)]
        _, slot = _argmax16_base(vwin, s_win * 16)
        _rmw_store(selgid_v, lanes, t, slot)
        vnew = jnp.where(lanes == slot - s_win * 16, jnp.float32(_NEG), vwin)
        gm_v[pl.ds(s_win * 16, 16)] = vnew
        _rmw_store(pvm_v, lanes, s_win, _max16(vnew))
        return c
    lax.fori_loop(0, SEL, pick_group, 0)

    for vi in range(SEL // 16 + 1):
        gvec = selgid_v[pl.ds(vi * 16, 16)]
        for l in range(16):
            j = vi * 16 + l
            if j >= SEL:
                break
            slot = gvec[l]
            ql = slot // SLOT_PITCH
            t = slot - ql * SLOT_PITCH
            tc = jnp.minimum(t, TILES - 1)
            pltpu.async_copy(
                raw_hbm.at[pl.ds((2 * b + ql) * 8, 8), pl.ds(tc * 128, 128)],
                grp_v.at[j], sem)
    for ql in range(2):
        pltpu.async_copy(
            raw_hbm.at[pl.ds((2 * b + ql) * 8, 8), pl.ds(TAIL0, 32)],
            tail_v.at[ql], sem)

    def drain_tile(j, c):
        pltpu.make_async_copy(raw_hbm.at[pl.ds(0, 8), pl.ds(0, 128)],
                              grp_v.at[j], sem).wait()
        return c
    lax.fori_loop(0, SEL, drain_tile, 0)
    for ql in range(2):
        pltpu.make_async_copy(raw_hbm.at[pl.ds(0, 8), pl.ds(TAIL0, 32)],
                              tail_v.at[ql], sem).wait()

    gbv_v[pl.ds(64, 16)] = jnp.full((16,), -2.0, jnp.float32)
    gbi_v[pl.ds(64, 16)] = jnp.full((16,), _BIG)

    def group_init(j, c):
        gvec = selgid_v[pl.ds((j // 16) * 16, 16)]
        slot = _select16([gvec[l] for l in range(16)], j - (j // 16) * 16)
        ql = slot // SLOT_PITCH
        t = slot - ql * SLOT_PITCH
        dummy = t == TILES
        fb = (ql * 8) * V + t * 128
        bias = jnp.where(dummy, jnp.float32(_NEG), jnp.float32(0.0))
        mv = jnp.full((16,), -2.0, jnp.float32)
        iv = jnp.full((16,), _BIG, jnp.int32)
        for s in range(8):
            scale = jnp.where(dummy, jnp.float32(0.0),
                              _select16(par_s, ql * 8 + s))
            for v in range(8):
                raw = grp_v[j, s, pl.ds(v * 16, 16)]
                val = raw * scale + bias
                grp_v[j, s, pl.ds(v * 16, 16)] = val
                fids = fb + s * V + v * 16 + lanes
                take = (val > mv) | ((val == mv) & (fids < iv))
                mv = jnp.where(take, val, mv)
                iv = jnp.where(take, fids, iv)
        bv, bi = _argmax16_pair(mv, iv)
        _rmw_store(gbv_v, lanes, j, bv)
        _rmw_store(gbi_v, lanes, j, bi)
        _rmw_store(fb_v, lanes, j, fb)
        return c
    lax.fori_loop(0, SEL, group_init, 0)

    for ql in range(2):
        fb = (ql * 8) * V + TAIL0
        mv = jnp.full((16,), -2.0, jnp.float32)
        iv = jnp.full((16,), _BIG, jnp.int32)
        for s in range(8):
            scale = par_s[ql * 8 + s]
            for v in range(2):
                val = tail_v[ql, s, pl.ds(v * 16, 16)] * scale
                tail_v[ql, s, pl.ds(v * 16, 16)] = val
                fids = fb + s * V + v * 16 + lanes
                take = (val > mv) | ((val == mv) & (fids < iv))
                mv = jnp.where(take, val, mv)
                iv = jnp.where(take, fids, iv)
        bv, bi = _argmax16_pair(mv, iv)
        _rmw_store(gbv_v, lanes, SEL + ql, bv)
        _rmw_store(gbi_v, lanes, SEL + ql, bi)
        _rmw_store(fb_v, lanes, SEL + ql, fb)

    def round_t(t, c):
        mv = jnp.full((16,), -2.0, jnp.float32)
        iv = jnp.full((16,), _BIG, jnp.int32)
        jv = jnp.full((16,), _BIG, jnp.int32)
        for k in range(5):
            v = gbv_v[pl.ds(k * 16, 16)]
            fi = gbi_v[pl.ds(k * 16, 16)]
            take = (v > mv) | ((v == mv) & (fi < iv))
            mv = jnp.where(take, v, mv)
            iv = jnp.where(take, fi, iv)
            jv = jnp.where(take, k * 16 + lanes, jv)
        m, fwin, jwin = _argmax16_triple(mv, iv, jv)

        _rmw_store(prb_v, lanes, t, m)
        _rmw_store(tok_v, lanes, t, fwin % V)
        _rmw_store(pidx_v, lanes, t, fwin // V)

        n = fwin // V
        col = fwin - n * V
        s_r = n - (n // 8) * 8
        fvec = fb_v[pl.ds((jwin // 16) * 16, 16)]
        fb = _select16([fvec[l] for l in range(16)], jwin - (jwin // 16) * 16)
        loc = col - (fb - (fb // V) * V)
        vr = loc // 16
        lpos = loc - vr * 16

        is_reg = jwin < SEL
        j_c = jnp.minimum(jwin, SEL - 1)
        ql_c = jnp.clip(jwin - SEL, 0, 1)

        lpos_r = jnp.where(is_reg, lpos, 16)
        lpos_t = jnp.where(is_reg, 16, lpos)
        vec = grp_v[j_c, s_r, pl.ds(vr * 16, 16)]
        grp_v[j_c, s_r, pl.ds(vr * 16, 16)] = jnp.where(
            lanes == lpos_r, jnp.float32(_NEG), vec)
        vr_t = jnp.minimum(vr, 1)
        vec2 = tail_v[ql_c, s_r, pl.ds(vr_t * 16, 16)]
        tail_v[ql_c, s_r, pl.ds(vr_t * 16, 16)] = jnp.where(
            lanes == lpos_t, jnp.float32(_NEG), vec2)

        mv1 = jnp.full((16,), -2.0, jnp.float32)
        iv1 = jnp.full((16,), _BIG, jnp.int32)
        for s in range(8):
            for v in range(8):
                val = grp_v[j_c, s, pl.ds(v * 16, 16)]
                fids = fb + s * V + v * 16 + lanes
                take = (val > mv1) | ((val == mv1) & (fids < iv1))
                mv1 = jnp.where(take, val, mv1)
                iv1 = jnp.where(take, fids, iv1)
        bv1, bi1 = _argmax16_pair(mv1, iv1)

        fb_t = (ql_c * 8) * V + TAIL0
        mv2 = jnp.full((16,), -2.0, jnp.float32)
        iv2 = jnp.full((16,), _BIG, jnp.int32)
        for s in range(8):
            for v in range(2):
                val = tail_v[ql_c, s, pl.ds(v * 16, 16)]
                fids = fb_t + s * V + v * 16 + lanes
                take = (val > mv2) | ((val == mv2) & (fids < iv2))
                mv2 = jnp.where(take, val, mv2)
                iv2 = jnp.where(take, fids, iv2)
        bv2, bi2 = _argmax16_pair(mv2, iv2)

        bv = jnp.where(is_reg, bv1, bv2)
        bi = jnp.where(is_reg, bi1, bi2)
        _rmw_store(gbv_v, lanes, jwin, bv)
        _rmw_store(gbi_v, lanes, jwin, bi)
        return c
    lax.fori_loop(0, K, round_t, 0)

    pltpu.sync_copy(tok_v, tok_hbm.at[b])
    pltpu.sync_copy(prb_v, prb_hbm.at[b])
    pltpu.sync_copy(pidx_v, pidx_hbm.at[b])


def _phase_b(sgm, parent_probs, raw2d):
    mesh = plsc.VectorSubcoreMesh(core_axis_name="c", subcore_axis_name="s")
    fn = functools.partial(
        pl.kernel,
        mesh=mesh,
        out_type=[
            jax.ShapeDtypeStruct((B, K), jnp.int32),
            jax.ShapeDtypeStruct((B, K), jnp.float32),
            jax.ShapeDtypeStruct((B, K), jnp.int32),
        ],
        scratch_types=[
            pltpu.VMEM((GMLEN,), jnp.float32),
            pltpu.VMEM((16,), jnp.float32),
            pltpu.VMEM((NPV * 16,), jnp.float32),
            pltpu.VMEM((80,), jnp.int32),
            pltpu.VMEM((80,), jnp.int32),
            pltpu.VMEM((SEL, 8, 128), jnp.float32),
            pltpu.VMEM((2, 8, 32), jnp.float32),
            pltpu.VMEM((80,), jnp.float32),
            pltpu.VMEM((80,), jnp.int32),
            pltpu.VMEM((K,), jnp.int32),
            pltpu.VMEM((K,), jnp.float32),
            pltpu.VMEM((K,), jnp.int32),
            pltpu.SemaphoreType.DMA,
        ],
    )(_phase_b_body)
    return fn(sgm, parent_probs, raw2d)


def kernel(sampled_probs, parent_probs, sample_k, sample_min_prob):
    del sample_k, sample_min_prob
    raw2d = sampled_probs.reshape(B * N, V)
    par2d = parent_probs.reshape(B * N, 1)
    sgm = _phase_a(raw2d, par2d).reshape(B * N // A_ROWS * SLOT_PITCH)
    tok, prb, pidx = _phase_b(sgm, parent_probs, raw2d)
    return tok, prb, pidx


# speedup vs baseline: 67.0909x; 1.0372x over previous
import functools

import jax
import jax.numpy as jnp
from jax import lax
from jax.experimental import pallas as pl
from jax.experimental.pallas import tpu as pltpu
from jax.experimental.pallas import tpu_sc as plsc

B, N, V = 32, 16, 100000
K = 64
SEL = 72
NSLOT = SEL + 2
TILES = V // 128
TAIL0 = TILES * 128
GPB = TILES + 1
SLOT_PITCH = 1024
NGV = 128
NPV = 8
A_ROWS = 8

_NEG = -1.0
_BIG = 0x7FFFFFFF


def _groupmax_body(x_ref, p_ref, o_ref):
    xs = x_ref[...] * p_ref[...]
    main = jnp.max(xs[:, :TAIL0].reshape(A_ROWS, TILES, 128), axis=(0, 2))
    tail = jnp.max(xs[:, TAIL0:])
    pad = jnp.full((SLOT_PITCH - GPB,), _NEG, jnp.float32)
    o_ref[...] = jnp.concatenate(
        [main, tail.reshape(1), pad]).reshape(1, 8, 128)


def _phase_a(raw2d, par2d):
    return pl.pallas_call(
        _groupmax_body,
        grid=(B * N // A_ROWS,),
        in_specs=[
            pl.BlockSpec((A_ROWS, V), lambda i: (i, 0)),
            pl.BlockSpec((A_ROWS, 1), lambda i: (i, 0)),
        ],
        out_specs=pl.BlockSpec((1, 8, 128), lambda i: (i, 0, 0)),
        out_shape=jax.ShapeDtypeStruct(
            (B * N // A_ROWS, 8, 128), jnp.float32),
    )(raw2d, par2d)


def _max16(v):
    m = v[0]
    for l in range(1, 16):
        m = jnp.maximum(m, v[l])
    return m


def _argmax16_base(v, base):
    m, mi = v[0], base
    for l in range(1, 16):
        better = v[l] > m
        m = jnp.where(better, v[l], m)
        mi = jnp.where(better, base + l, mi)
    return m, mi


def _argmax16_pair(mv, iv):
    m, mi = mv[0], iv[0]
    for l in range(1, 16):
        better = (mv[l] > m) | ((mv[l] == m) & (iv[l] < mi))
        m = jnp.where(better, mv[l], m)
        mi = jnp.where(better, iv[l], mi)
    return m, mi


def _argmax16_triple(mv, iv, jv):
    m, mi, mj = mv[0], iv[0], jv[0]
    for l in range(1, 16):
        better = (mv[l] > m) | ((mv[l] == m) & (iv[l] < mi))
        m = jnp.where(better, mv[l], m)
        mi = jnp.where(better, iv[l], mi)
        mj = jnp.where(better, jv[l], mj)
    return m, mi, mj


def _select16(vals, sel):
    acc = vals[0]
    for l in range(1, 16):
        acc = jnp.where(sel == l, vals[l], acc)
    return acc


def _rmw_store(ref, lanes, slot, val):
    off = (slot // 16) * 16
    vec = ref[pl.ds(off, 16)]
    ref[pl.ds(off, 16)] = jnp.where(lanes == slot - off, val, vec)


def _phase_b_body(sgm_hbm, par_hbm, raw_hbm,
                  tok_hbm, prb_hbm, pidx_hbm,
                  gm_v, par_v, pvm_v, selgid_v, fb_v, grp_v, tail_v,
                  gbv_v, gbi_v, tok_v, prb_v, pidx_v, sem):
    b = lax.axis_index("s") * 2 + lax.axis_index("c")
    lanes = lax.iota(jnp.int32, 16)

    pltpu.sync_copy(sgm_hbm.at[2 * b], gm_v.at[0])
    pltpu.sync_copy(sgm_hbm.at[2 * b + 1], gm_v.at[1])
    pltpu.sync_copy(par_hbm.at[b], par_v)
    pvec = par_v[pl.ds(0, 16)]
    par_s = [pvec[l] for l in range(16)]

    def vreg_max(i, c):
        ql = i // 64
        r = i - ql * 64
        s = r // 8
        v = r - s * 8
        _rmw_store(pvm_v, lanes, i, _max16(gm_v[ql, s, pl.ds(v * 16, 16)]))
        return c
    lax.fori_loop(0, NGV, vreg_max, 0)

    def pick_group(t, c):
        mv = jnp.full((16,), -2.0, jnp.float32)
        sv = jnp.full((16,), _BIG, jnp.int32)
        for k in range(NPV):
            v = pvm_v[pl.ds(k * 16, 16)]
            take = v > mv
            mv = jnp.where(take, v, mv)
            sv = jnp.where(take, k * 16 + lanes, sv)
        _, kwin = _argmax16_pair(mv, sv)
        ql = kwin // 64
        r = kwin - ql * 64
        s = r // 8
        v = r - s * 8
        vwin = gm_v[ql, s, pl.ds(v * 16, 16)]
        _, slot = _argmax16_base(vwin, kwin * 16)
        _rmw_store(selgid_v, lanes, t, slot)
        vnew = jnp.where(lanes == slot - kwin * 16, jnp.float32(_NEG), vwin)
        gm_v[ql, s, pl.ds(v * 16, 16)] = vnew
        _rmw_store(pvm_v, lanes, kwin, _max16(vnew))
        return c
    lax.fori_loop(0, SEL, pick_group, 0)

    for vi in range(SEL // 16 + 1):
        gvec = selgid_v[pl.ds(vi * 16, 16)]
        for l in range(16):
            j = vi * 16 + l
            if j >= SEL:
                break
            slot = gvec[l]
            ql = slot // SLOT_PITCH
            t = slot - ql * SLOT_PITCH
            tc = jnp.minimum(t, TILES - 1)
            pltpu.async_copy(
                raw_hbm.at[pl.ds((2 * b + ql) * 8, 8), pl.ds(tc * 128, 128)],
                grp_v.at[j], sem)
    for ql in range(2):
        pltpu.async_copy(
            raw_hbm.at[pl.ds((2 * b + ql) * 8, 8), pl.ds(TAIL0, 32)],
            tail_v.at[ql], sem)

    def drain_tile(j, c):
        pltpu.make_async_copy(raw_hbm.at[pl.ds(0, 8), pl.ds(0, 128)],
                              grp_v.at[j], sem).wait()
        return c
    lax.fori_loop(0, SEL, drain_tile, 0)
    for ql in range(2):
        pltpu.make_async_copy(raw_hbm.at[pl.ds(0, 8), pl.ds(TAIL0, 32)],
                              tail_v.at[ql], sem).wait()

    gbv_v[pl.ds(64, 16)] = jnp.full((16,), -2.0, jnp.float32)
    gbi_v[pl.ds(64, 16)] = jnp.full((16,), _BIG)

    def group_init(j, c):
        gvec = selgid_v[pl.ds((j // 16) * 16, 16)]
        slot = _select16([gvec[l] for l in range(16)], j - (j // 16) * 16)
        ql = slot // SLOT_PITCH
        t = slot - ql * SLOT_PITCH
        dummy = t == TILES
        fb = (ql * 8) * V + t * 128
        bias = jnp.where(dummy, jnp.float32(_NEG), jnp.float32(0.0))
        mv = jnp.full((16,), -2.0, jnp.float32)
        iv = jnp.full((16,), _BIG, jnp.int32)
        for s in range(8):
            scale = jnp.where(dummy, jnp.float32(0.0),
                              _select16(par_s, ql * 8 + s))
            for v in range(8):
                raw = grp_v[j, s, pl.ds(v * 16, 16)]
                val = raw * scale + bias
                grp_v[j, s, pl.ds(v * 16, 16)] = val
                fids = fb + s * V + v * 16 + lanes
                take = (val > mv) | ((val == mv) & (fids < iv))
                mv = jnp.where(take, val, mv)
                iv = jnp.where(take, fids, iv)
        bv, bi = _argmax16_pair(mv, iv)
        _rmw_store(gbv_v, lanes, j, bv)
        _rmw_store(gbi_v, lanes, j, bi)
        _rmw_store(fb_v, lanes, j, fb)
        return c
    lax.fori_loop(0, SEL, group_init, 0)

    neg16 = jnp.full((16,), _NEG, jnp.float32)
    for ql in range(2):
        fb = (ql * 8) * V + TAIL0
        mv = jnp.full((16,), -2.0, jnp.float32)
        iv = jnp.full((16,), _BIG, jnp.int32)
        for s in range(8):
            scale = par_s[ql * 8 + s]
            for v in range(8):
                if v < 2:
                    val = tail_v[ql, s, pl.ds(v * 16, 16)] * scale
                else:
                    val = neg16
                grp_v[SEL + ql, s, pl.ds(v * 16, 16)] = val
                fids = fb + s * V + v * 16 + lanes
                take = (val > mv) | ((val == mv) & (fids < iv))
                mv = jnp.where(take, val, mv)
                iv = jnp.where(take, fids, iv)
        bv, bi = _argmax16_pair(mv, iv)
        _rmw_store(gbv_v, lanes, SEL + ql, bv)
        _rmw_store(gbi_v, lanes, SEL + ql, bi)
        _rmw_store(fb_v, lanes, SEL + ql, fb)

    def round_t(t, c):
        mv = jnp.full((16,), -2.0, jnp.float32)
        iv = jnp.full((16,), _BIG, jnp.int32)
        jv = jnp.full((16,), _BIG, jnp.int32)
        for k in range(5):
            v = gbv_v[pl.ds(k * 16, 16)]
            fi = gbi_v[pl.ds(k * 16, 16)]
            take = (v > mv) | ((v == mv) & (fi < iv))
            mv = jnp.where(take, v, mv)
            iv = jnp.where(take, fi, iv)
            jv = jnp.where(take, k * 16 + lanes, jv)
        m, fwin, jwin = _argmax16_triple(mv, iv, jv)

        _rmw_store(prb_v, lanes, t, m)
        _rmw_store(tok_v, lanes, t, fwin % V)
        _rmw_store(pidx_v, lanes, t, fwin // V)

        n = fwin // V
        col = fwin - n * V
        s_r = n - (n // 8) * 8
        fvec = fb_v[pl.ds((jwin // 16) * 16, 16)]
        fb = _select16([fvec[l] for l in range(16)], jwin - (jwin // 16) * 16)
        loc = col - (fb - (fb // V) * V)
        vr = loc // 16
        lpos = loc - vr * 16

        vec = grp_v[jwin, s_r, pl.ds(vr * 16, 16)]
        grp_v[jwin, s_r, pl.ds(vr * 16, 16)] = jnp.where(
            lanes == lpos, jnp.float32(_NEG), vec)

        mv1 = jnp.full((16,), -2.0, jnp.float32)
        iv1 = jnp.full((16,), _BIG, jnp.int32)
        for s in range(8):
            for v in range(8):
                val = grp_v[jwin, s, pl.ds(v * 16, 16)]
                fids = fb + s * V + v * 16 + lanes
                take = (val > mv1) | ((val == mv1) & (fids < iv1))
                mv1 = jnp.where(take, val, mv1)
                iv1 = jnp.where(take, fids, iv1)
        bv, bi = _argmax16_pair(mv1, iv1)
        _rmw_store(gbv_v, lanes, jwin, bv)
        _rmw_store(gbi_v, lanes, jwin, bi)
        return c
    lax.fori_loop(0, K, round_t, 0)

    pltpu.sync_copy(tok_v, tok_hbm.at[b])
    pltpu.sync_copy(prb_v, prb_hbm.at[b])
    pltpu.sync_copy(pidx_v, pidx_hbm.at[b])


def _phase_b(sgm, parent_probs, raw2d):
    mesh = plsc.VectorSubcoreMesh(core_axis_name="c", subcore_axis_name="s")
    fn = functools.partial(
        pl.kernel,
        mesh=mesh,
        out_type=[
            jax.ShapeDtypeStruct((B, K), jnp.int32),
            jax.ShapeDtypeStruct((B, K), jnp.float32),
            jax.ShapeDtypeStruct((B, K), jnp.int32),
        ],
        scratch_types=[
            pltpu.VMEM((2, 8, 128), jnp.float32),
            pltpu.VMEM((16,), jnp.float32),
            pltpu.VMEM((NPV * 16,), jnp.float32),
            pltpu.VMEM((80,), jnp.int32),
            pltpu.VMEM((80,), jnp.int32),
            pltpu.VMEM((NSLOT, 8, 128), jnp.float32),
            pltpu.VMEM((2, 8, 32), jnp.float32),
            pltpu.VMEM((80,), jnp.float32),
            pltpu.VMEM((80,), jnp.int32),
            pltpu.VMEM((K,), jnp.int32),
            pltpu.VMEM((K,), jnp.float32),
            pltpu.VMEM((K,), jnp.int32),
            pltpu.SemaphoreType.DMA,
        ],
    )(_phase_b_body)
    return fn(sgm, parent_probs, raw2d)


def kernel(sampled_probs, parent_probs, sample_k, sample_min_prob):
    del sample_k, sample_min_prob
    raw2d = sampled_probs.reshape(B * N, V)
    par2d = parent_probs.reshape(B * N, 1)
    sgm = _phase_a(raw2d, par2d)
    tok, prb, pidx = _phase_b(sgm, parent_probs, raw2d)
    return tok, prb, pidx


# speedup vs baseline: 72.8160x; 1.0853x over previous
import functools

import jax
import jax.numpy as jnp
from jax import lax
from jax.experimental import pallas as pl
from jax.experimental.pallas import tpu as pltpu
from jax.experimental.pallas import tpu_sc as plsc

B, N, V = 32, 16, 100000
K = 64
SEL = 72
NSLOT = SEL + 2
TILES = V // 128
TAIL0 = TILES * 128
GPB = TILES + 1
SLOT_PITCH = 1024
NGV = 128
NPV = 8
A_ROWS = 8

_NEG = -1.0
_BIG = 0x7FFFFFFF


def _groupmax_body(x_ref, p_ref, o_ref):
    xs = x_ref[...] * p_ref[...]
    main = jnp.max(xs[:, :TAIL0].reshape(A_ROWS, TILES, 128), axis=(0, 2))
    tail = jnp.max(xs[:, TAIL0:])
    pad = jnp.full((SLOT_PITCH - GPB,), _NEG, jnp.float32)
    o_ref[...] = jnp.concatenate(
        [main, tail.reshape(1), pad]).reshape(1, 8, 128)


def _phase_a(raw2d, par2d):
    return pl.pallas_call(
        _groupmax_body,
        grid=(B * N // A_ROWS,),
        in_specs=[
            pl.BlockSpec((A_ROWS, V), lambda i: (i, 0)),
            pl.BlockSpec((A_ROWS, 1), lambda i: (i, 0)),
        ],
        out_specs=pl.BlockSpec((1, 8, 128), lambda i: (i, 0, 0)),
        out_shape=jax.ShapeDtypeStruct(
            (B * N // A_ROWS, 8, 128), jnp.float32),
    )(raw2d, par2d)


def _max16(v):
    m = v[0]
    for l in range(1, 16):
        m = jnp.maximum(m, v[l])
    return m


def _argmax16_base(v, base):
    m, mi = v[0], base
    for l in range(1, 16):
        better = v[l] > m
        m = jnp.where(better, v[l], m)
        mi = jnp.where(better, base + l, mi)
    return m, mi


def _argmax16_pair(mv, iv):
    m, mi = mv[0], iv[0]
    for l in range(1, 16):
        better = (mv[l] > m) | ((mv[l] == m) & (iv[l] < mi))
        m = jnp.where(better, mv[l], m)
        mi = jnp.where(better, iv[l], mi)
    return m, mi


def _argmax16_triple(mv, iv, jv):
    m, mi, mj = mv[0], iv[0], jv[0]
    for l in range(1, 16):
        better = (mv[l] > m) | ((mv[l] == m) & (iv[l] < mi))
        m = jnp.where(better, mv[l], m)
        mi = jnp.where(better, iv[l], mi)
        mj = jnp.where(better, jv[l], mj)
    return m, mi, mj


def _select16(vals, sel):
    acc = vals[0]
    for l in range(1, 16):
        acc = jnp.where(sel == l, vals[l], acc)
    return acc


def _rmw_store(ref, lanes, slot, val):
    off = (slot // 16) * 16
    vec = ref[pl.ds(off, 16)]
    ref[pl.ds(off, 16)] = jnp.where(lanes == slot - off, val, vec)


def _phase_b_body(sgm_hbm, par_hbm, raw_hbm,
                  tok_hbm, prb_hbm, pidx_hbm,
                  gm_v, par_v, pvm_v, selgid_v, fb_v, grp_v, tail_v,
                  gbv_v, gbi_v, tok_v, prb_v, pidx_v, sem):
    b = lax.axis_index("s") * 2 + lax.axis_index("c")
    lanes = lax.iota(jnp.int32, 16)

    pltpu.sync_copy(sgm_hbm.at[2 * b], gm_v.at[0])
    pltpu.sync_copy(sgm_hbm.at[2 * b + 1], gm_v.at[1])
    pltpu.sync_copy(par_hbm.at[b], par_v)
    pvec = par_v[pl.ds(0, 16)]
    par_s = [pvec[l] for l in range(16)]

    def vreg_max(i, c):
        ql = i // 64
        r = i - ql * 64
        s = r // 8
        v = r - s * 8
        _rmw_store(pvm_v, lanes, i, _max16(gm_v[ql, s, pl.ds(v * 16, 16)]))
        return c
    lax.fori_loop(0, NGV, vreg_max, 0)

    def pick_group(t, c):
        mv = jnp.full((16,), -2.0, jnp.float32)
        sv = jnp.full((16,), _BIG, jnp.int32)
        for k in range(NPV):
            v = pvm_v[pl.ds(k * 16, 16)]
            take = v > mv
            mv = jnp.where(take, v, mv)
            sv = jnp.where(take, k * 16 + lanes, sv)
        _, kwin = _argmax16_pair(mv, sv)
        ql = kwin // 64
        r = kwin - ql * 64
        s = r // 8
        v = r - s * 8
        vwin = gm_v[ql, s, pl.ds(v * 16, 16)]
        _, slot = _argmax16_base(vwin, kwin * 16)
        _rmw_store(selgid_v, lanes, t, slot)
        vnew = jnp.where(lanes == slot - kwin * 16, jnp.float32(_NEG), vwin)
        gm_v[ql, s, pl.ds(v * 16, 16)] = vnew
        _rmw_store(pvm_v, lanes, kwin, _max16(vnew))
        tt = slot - ql * SLOT_PITCH
        tc = jnp.minimum(tt, TILES - 1)
        pltpu.async_copy(
            raw_hbm.at[pl.ds((2 * b + ql) * 8, 8), pl.ds(tc * 128, 128)],
            grp_v.at[t], sem)
        return c
    lax.fori_loop(0, SEL, pick_group, 0)

    for ql in range(2):
        pltpu.async_copy(
            raw_hbm.at[pl.ds((2 * b + ql) * 8, 8), pl.ds(TAIL0, 32)],
            tail_v.at[ql], sem)

    def drain_tile(j, c):
        pltpu.make_async_copy(raw_hbm.at[pl.ds(0, 8), pl.ds(0, 128)],
                              grp_v.at[j], sem).wait()
        return c
    lax.fori_loop(0, SEL, drain_tile, 0)
    for ql in range(2):
        pltpu.make_async_copy(raw_hbm.at[pl.ds(0, 8), pl.ds(TAIL0, 32)],
                              tail_v.at[ql], sem).wait()

    gbv_v[pl.ds(64, 16)] = jnp.full((16,), -2.0, jnp.float32)
    gbi_v[pl.ds(64, 16)] = jnp.full((16,), _BIG)

    def group_init(j, c):
        gvec = selgid_v[pl.ds((j // 16) * 16, 16)]
        slot = _select16([gvec[l] for l in range(16)], j - (j // 16) * 16)
        ql = slot // SLOT_PITCH
        t = slot - ql * SLOT_PITCH
        dummy = t == TILES
        fb = (ql * 8) * V + t * 128
        bias = jnp.where(dummy, jnp.float32(_NEG), jnp.float32(0.0))
        mv = jnp.full((16,), -2.0, jnp.float32)
        iv = jnp.full((16,), _BIG, jnp.int32)
        zero = jnp.float32(0.0)
        for s in range(8):
            scale = jnp.where(dummy, zero,
                              jnp.where(ql == 0, par_s[s], par_s[8 + s]))
            for v in range(8):
                raw = grp_v[j, s, pl.ds(v * 16, 16)]
                val = raw * scale + bias
                grp_v[j, s, pl.ds(v * 16, 16)] = val
                fids = fb + s * V + v * 16 + lanes
                take = val > mv
                mv = jnp.where(take, val, mv)
                iv = jnp.where(take, fids, iv)
        bv, bi = _argmax16_pair(mv, iv)
        _rmw_store(gbv_v, lanes, j, bv)
        _rmw_store(gbi_v, lanes, j, bi)
        _rmw_store(fb_v, lanes, j, fb)
        return c
    lax.fori_loop(0, SEL, group_init, 0)

    neg16 = jnp.full((16,), _NEG, jnp.float32)
    for ql in range(2):
        fb = (ql * 8) * V + TAIL0
        mv = jnp.full((16,), -2.0, jnp.float32)
        iv = jnp.full((16,), _BIG, jnp.int32)
        for s in range(8):
            scale = par_s[ql * 8 + s]
            for v in range(8):
                if v < 2:
                    val = tail_v[ql, s, pl.ds(v * 16, 16)] * scale
                else:
                    val = neg16
                grp_v[SEL + ql, s, pl.ds(v * 16, 16)] = val
                fids = fb + s * V + v * 16 + lanes
                take = val > mv
                mv = jnp.where(take, val, mv)
                iv = jnp.where(take, fids, iv)
        bv, bi = _argmax16_pair(mv, iv)
        _rmw_store(gbv_v, lanes, SEL + ql, bv)
        _rmw_store(gbi_v, lanes, SEL + ql, bi)
        _rmw_store(fb_v, lanes, SEL + ql, fb)

    def round_t(t, c):
        mv = jnp.full((16,), -2.0, jnp.float32)
        iv = jnp.full((16,), _BIG, jnp.int32)
        jv = jnp.full((16,), _BIG, jnp.int32)
        for k in range(5):
            v = gbv_v[pl.ds(k * 16, 16)]
            fi = gbi_v[pl.ds(k * 16, 16)]
            take = (v > mv) | ((v == mv) & (fi < iv))
            mv = jnp.where(take, v, mv)
            iv = jnp.where(take, fi, iv)
            jv = jnp.where(take, k * 16 + lanes, jv)
        m, fwin, jwin = _argmax16_triple(mv, iv, jv)

        _rmw_store(prb_v, lanes, t, m)
        _rmw_store(tok_v, lanes, t, fwin % V)
        _rmw_store(pidx_v, lanes, t, fwin // V)

        n = fwin // V
        col = fwin - n * V
        s_r = n - (n // 8) * 8
        fvec = fb_v[pl.ds((jwin // 16) * 16, 16)]
        fb = _select16([fvec[l] for l in range(16)], jwin - (jwin // 16) * 16)
        loc = col - (fb - (fb // V) * V)
        vr = loc // 16
        lpos = loc - vr * 16

        vec = grp_v[jwin, s_r, pl.ds(vr * 16, 16)]
        grp_v[jwin, s_r, pl.ds(vr * 16, 16)] = jnp.where(
            lanes == lpos, jnp.float32(_NEG), vec)

        mv1 = jnp.full((16,), -2.0, jnp.float32)
        iv1 = jnp.full((16,), _BIG, jnp.int32)
        for s in range(8):
            for v in range(8):
                val = grp_v[jwin, s, pl.ds(v * 16, 16)]
                fids = fb + s * V + v * 16 + lanes
                take = val > mv1
                mv1 = jnp.where(take, val, mv1)
                iv1 = jnp.where(take, fids, iv1)
        bv, bi = _argmax16_pair(mv1, iv1)
        _rmw_store(gbv_v, lanes, jwin, bv)
        _rmw_store(gbi_v, lanes, jwin, bi)
        return c
    lax.fori_loop(0, K, round_t, 0)

    pltpu.sync_copy(tok_v, tok_hbm.at[b])
    pltpu.sync_copy(prb_v, prb_hbm.at[b])
    pltpu.sync_copy(pidx_v, pidx_hbm.at[b])


def _phase_b(sgm, parent_probs, raw2d):
    mesh = plsc.VectorSubcoreMesh(core_axis_name="c", subcore_axis_name="s")
    fn = functools.partial(
        pl.kernel,
        mesh=mesh,
        out_type=[
            jax.ShapeDtypeStruct((B, K), jnp.int32),
            jax.ShapeDtypeStruct((B, K), jnp.float32),
            jax.ShapeDtypeStruct((B, K), jnp.int32),
        ],
        scratch_types=[
            pltpu.VMEM((2, 8, 128), jnp.float32),
            pltpu.VMEM((16,), jnp.float32),
            pltpu.VMEM((NPV * 16,), jnp.float32),
            pltpu.VMEM((80,), jnp.int32),
            pltpu.VMEM((80,), jnp.int32),
            pltpu.VMEM((NSLOT, 8, 128), jnp.float32),
            pltpu.VMEM((2, 8, 32), jnp.float32),
            pltpu.VMEM((80,), jnp.float32),
            pltpu.VMEM((80,), jnp.int32),
            pltpu.VMEM((K,), jnp.int32),
            pltpu.VMEM((K,), jnp.float32),
            pltpu.VMEM((K,), jnp.int32),
            pltpu.SemaphoreType.DMA,
        ],
    )(_phase_b_body)
    return fn(sgm, parent_probs, raw2d)


def kernel(sampled_probs, parent_probs, sample_k, sample_min_prob):
    del sample_k, sample_min_prob
    raw2d = sampled_probs.reshape(B * N, V)
    par2d = parent_probs.reshape(B * N, 1)
    sgm = _phase_a(raw2d, par2d)
    tok, prb, pidx = _phase_b(sgm, parent_probs, raw2d)
    return tok, prb, pidx


# speedup vs baseline: 83.1964x; 1.1426x over previous
import functools

import jax
import jax.numpy as jnp
from jax import lax
from jax.experimental import pallas as pl
from jax.experimental.pallas import tpu as pltpu
from jax.experimental.pallas import tpu_sc as plsc

B, N, V = 32, 16, 100000
K = 64
SEL = 72
NSLOT = SEL + 2
TILES = V // 128
TAIL0 = TILES * 128
GPB = TILES + 1
SLOT_PITCH = 1024
NGV = 128
NPV = 8
A_ROWS = 16

_NEG = -1.0
_BIG = 0x7FFFFFFF


def _groupmax_body(x_ref, p_ref, o_ref):
    xs = x_ref[...] * p_ref[...]
    main = jnp.max(xs[:, :TAIL0].reshape(A_ROWS, TILES, 128), axis=(0, 2))
    tail = jnp.max(xs[:, TAIL0:])
    pad = jnp.full((SLOT_PITCH - GPB,), _NEG, jnp.float32)
    o_ref[...] = jnp.concatenate(
        [main, tail.reshape(1), pad]).reshape(1, 8, 128)


def _phase_a(raw2d, par2d):
    return pl.pallas_call(
        _groupmax_body,
        grid=(B * N // A_ROWS,),
        in_specs=[
            pl.BlockSpec((A_ROWS, V), lambda i: (i, 0)),
            pl.BlockSpec((A_ROWS, 1), lambda i: (i, 0)),
        ],
        out_specs=pl.BlockSpec((1, 8, 128), lambda i: (i, 0, 0)),
        out_shape=jax.ShapeDtypeStruct(
            (B * N // A_ROWS, 8, 128), jnp.float32),
    )(raw2d, par2d)


def _max16(v):
    m = v[0]
    for l in range(1, 16):
        m = jnp.maximum(m, v[l])
    return m


def _argmax16_base(v, base):
    m, mi = v[0], base
    for l in range(1, 16):
        better = v[l] > m
        m = jnp.where(better, v[l], m)
        mi = jnp.where(better, base + l, mi)
    return m, mi


def _argmax16_pair(mv, iv):
    m, mi = mv[0], iv[0]
    for l in range(1, 16):
        better = (mv[l] > m) | ((mv[l] == m) & (iv[l] < mi))
        m = jnp.where(better, mv[l], m)
        mi = jnp.where(better, iv[l], mi)
    return m, mi


def _argmax16_triple(mv, iv, jv):
    m, mi, mj = mv[0], iv[0], jv[0]
    for l in range(1, 16):
        better = (mv[l] > m) | ((mv[l] == m) & (iv[l] < mi))
        m = jnp.where(better, mv[l], m)
        mi = jnp.where(better, iv[l], mi)
        mj = jnp.where(better, jv[l], mj)
    return m, mi, mj


def _select16(vals, sel):
    acc = vals[0]
    for l in range(1, 16):
        acc = jnp.where(sel == l, vals[l], acc)
    return acc


def _rmw_store(ref, lanes, slot, val):
    off = (slot // 16) * 16
    vec = ref[pl.ds(off, 16)]
    ref[pl.ds(off, 16)] = jnp.where(lanes == slot - off, val, vec)


def _phase_b_body(sgm_hbm, par_hbm, raw_hbm,
                  tok_hbm, prb_hbm, pidx_hbm,
                  gm_v, par_v, pvm_v, selgid_v, fb_v, grp_v, tail_v,
                  gbv_v, gbi_v, tok_v, prb_v, pidx_v, sem):
    b = lax.axis_index("s") * 2 + lax.axis_index("c")
    lanes = lax.iota(jnp.int32, 16)

    pltpu.sync_copy(sgm_hbm.at[2 * b], gm_v.at[0])
    pltpu.sync_copy(sgm_hbm.at[2 * b + 1], gm_v.at[1])
    pltpu.sync_copy(par_hbm.at[b], par_v)
    pvec = par_v[pl.ds(0, 16)]
    par_s = [pvec[l] for l in range(16)]

    def vreg_max(i, c):
        ql = i // 64
        r = i - ql * 64
        s = r // 8
        v = r - s * 8
        _rmw_store(pvm_v, lanes, i, _max16(gm_v[ql, s, pl.ds(v * 16, 16)]))
        return c
    lax.fori_loop(0, NGV, vreg_max, 0)

    def pick_group(t, c):
        mv = jnp.full((16,), -2.0, jnp.float32)
        sv = jnp.full((16,), _BIG, jnp.int32)
        for k in range(NPV):
            v = pvm_v[pl.ds(k * 16, 16)]
            take = v > mv
            mv = jnp.where(take, v, mv)
            sv = jnp.where(take, k * 16 + lanes, sv)
        _, kwin = _argmax16_pair(mv, sv)
        ql = kwin // 64
        r = kwin - ql * 64
        s = r // 8
        v = r - s * 8
        vwin = gm_v[ql, s, pl.ds(v * 16, 16)]
        _, slot = _argmax16_base(vwin, kwin * 16)
        _rmw_store(selgid_v, lanes, t, slot)
        vnew = jnp.where(lanes == slot - kwin * 16, jnp.float32(_NEG), vwin)
        gm_v[ql, s, pl.ds(v * 16, 16)] = vnew
        _rmw_store(pvm_v, lanes, kwin, _max16(vnew))
        tt = slot - ql * SLOT_PITCH
        tc = jnp.minimum(tt, TILES - 1)
        pltpu.async_copy(
            raw_hbm.at[pl.ds((2 * b + ql) * 8, 8), pl.ds(tc * 128, 128)],
            grp_v.at[t], sem)
        return c
    lax.fori_loop(0, SEL, pick_group, 0)

    for ql in range(2):
        pltpu.async_copy(
            raw_hbm.at[pl.ds((2 * b + ql) * 8, 8), pl.ds(TAIL0, 32)],
            tail_v.at[ql], sem)

    def drain_tile(j, c):
        pltpu.make_async_copy(raw_hbm.at[pl.ds(0, 8), pl.ds(0, 128)],
                              grp_v.at[j], sem).wait()
        return c
    lax.fori_loop(0, SEL, drain_tile, 0)
    for ql in range(2):
        pltpu.make_async_copy(raw_hbm.at[pl.ds(0, 8), pl.ds(TAIL0, 32)],
                              tail_v.at[ql], sem).wait()

    gbv_v[pl.ds(64, 16)] = jnp.full((16,), -2.0, jnp.float32)
    gbi_v[pl.ds(64, 16)] = jnp.full((16,), _BIG)

    def group_init(j, c):
        gvec = selgid_v[pl.ds((j // 16) * 16, 16)]
        slot = _select16([gvec[l] for l in range(16)], j - (j // 16) * 16)
        ql = slot // SLOT_PITCH
        t = slot - ql * SLOT_PITCH
        dummy = t == TILES
        fb = (ql * 8) * V + t * 128
        bias = jnp.where(dummy, jnp.float32(_NEG), jnp.float32(0.0))
        mv = jnp.full((16,), -2.0, jnp.float32)
        iv = jnp.full((16,), _BIG, jnp.int32)
        zero = jnp.float32(0.0)
        for s in range(8):
            scale = jnp.where(dummy, zero,
                              jnp.where(ql == 0, par_s[s], par_s[8 + s]))
            for v in range(8):
                raw = grp_v[j, s, pl.ds(v * 16, 16)]
                val = raw * scale + bias
                grp_v[j, s, pl.ds(v * 16, 16)] = val
                fids = fb + s * V + v * 16 + lanes
                take = val > mv
                mv = jnp.where(take, val, mv)
                iv = jnp.where(take, fids, iv)
        bv, bi = _argmax16_pair(mv, iv)
        _rmw_store(gbv_v, lanes, j, bv)
        _rmw_store(gbi_v, lanes, j, bi)
        _rmw_store(fb_v, lanes, j, fb)
        return c
    lax.fori_loop(0, SEL, group_init, 0)

    neg16 = jnp.full((16,), _NEG, jnp.float32)
    for ql in range(2):
        fb = (ql * 8) * V + TAIL0
        mv = jnp.full((16,), -2.0, jnp.float32)
        iv = jnp.full((16,), _BIG, jnp.int32)
        for s in range(8):
            scale = par_s[ql * 8 + s]
            for v in range(8):
                if v < 2:
                    val = tail_v[ql, s, pl.ds(v * 16, 16)] * scale
                else:
                    val = neg16
                grp_v[SEL + ql, s, pl.ds(v * 16, 16)] = val
                fids = fb + s * V + v * 16 + lanes
                take = val > mv
                mv = jnp.where(take, val, mv)
                iv = jnp.where(take, fids, iv)
        bv, bi = _argmax16_pair(mv, iv)
        _rmw_store(gbv_v, lanes, SEL + ql, bv)
        _rmw_store(gbi_v, lanes, SEL + ql, bi)
        _rmw_store(fb_v, lanes, SEL + ql, fb)

    def round_t(t, c):
        mv = jnp.full((16,), -2.0, jnp.float32)
        iv = jnp.full((16,), _BIG, jnp.int32)
        jv = jnp.full((16,), _BIG, jnp.int32)
        for k in range(5):
            v = gbv_v[pl.ds(k * 16, 16)]
            fi = gbi_v[pl.ds(k * 16, 16)]
            take = (v > mv) | ((v == mv) & (fi < iv))
            mv = jnp.where(take, v, mv)
            iv = jnp.where(take, fi, iv)
            jv = jnp.where(take, k * 16 + lanes, jv)
        m, fwin, jwin = _argmax16_triple(mv, iv, jv)

        _rmw_store(prb_v, lanes, t, m)
        _rmw_store(tok_v, lanes, t, fwin % V)
        _rmw_store(pidx_v, lanes, t, fwin // V)

        n = fwin // V
        col = fwin - n * V
        s_r = n - (n // 8) * 8
        fvec = fb_v[pl.ds((jwin // 16) * 16, 16)]
        fb = _select16([fvec[l] for l in range(16)], jwin - (jwin // 16) * 16)
        loc = col - (fb - (fb // V) * V)
        vr = loc // 16
        lpos = loc - vr * 16

        vec = grp_v[jwin, s_r, pl.ds(vr * 16, 16)]
        grp_v[jwin, s_r, pl.ds(vr * 16, 16)] = jnp.where(
            lanes == lpos, jnp.float32(_NEG), vec)

        mv1 = jnp.full((16,), -2.0, jnp.float32)
        iv1 = jnp.full((16,), _BIG, jnp.int32)
        for s in range(8):
            for v in range(8):
                val = grp_v[jwin, s, pl.ds(v * 16, 16)]
                fids = fb + s * V + v * 16 + lanes
                take = val > mv1
                mv1 = jnp.where(take, val, mv1)
                iv1 = jnp.where(take, fids, iv1)
        bv, bi = _argmax16_pair(mv1, iv1)
        _rmw_store(gbv_v, lanes, jwin, bv)
        _rmw_store(gbi_v, lanes, jwin, bi)
        return c
    lax.fori_loop(0, K, round_t, 0)

    pltpu.sync_copy(tok_v, tok_hbm.at[b])
    pltpu.sync_copy(prb_v, prb_hbm.at[b])
    pltpu.sync_copy(pidx_v, pidx_hbm.at[b])


def _phase_b(sgm, parent_probs, raw2d):
    mesh = plsc.VectorSubcoreMesh(core_axis_name="c", subcore_axis_name="s")
    fn = functools.partial(
        pl.kernel,
        mesh=mesh,
        out_type=[
            jax.ShapeDtypeStruct((B, K), jnp.int32),
            jax.ShapeDtypeStruct((B, K), jnp.float32),
            jax.ShapeDtypeStruct((B, K), jnp.int32),
        ],
        scratch_types=[
            pltpu.VMEM((2, 8, 128), jnp.float32),
            pltpu.VMEM((16,), jnp.float32),
            pltpu.VMEM((NPV * 16,), jnp.float32),
            pltpu.VMEM((80,), jnp.int32),
            pltpu.VMEM((80,), jnp.int32),
            pltpu.VMEM((NSLOT, 8, 128), jnp.float32),
            pltpu.VMEM((2, 8, 32), jnp.float32),
            pltpu.VMEM((80,), jnp.float32),
            pltpu.VMEM((80,), jnp.int32),
            pltpu.VMEM((K,), jnp.int32),
            pltpu.VMEM((K,), jnp.float32),
            pltpu.VMEM((K,), jnp.int32),
            pltpu.SemaphoreType.DMA,
        ],
    )(_phase_b_body)
    return fn(sgm, parent_probs, raw2d)


def kernel(sampled_probs, parent_probs, sample_k, sample_min_prob):
    del sample_k, sample_min_prob
    raw2d = sampled_probs.reshape(B * N, V)
    par2d = parent_probs.reshape(B * N, 1)
    sgm = _phase_a(raw2d, par2d)
    tok, prb, pidx = _phase_b(sgm, parent_probs, raw2d)
    return tok, prb, pidx


# speedup vs baseline: 85.7008x; 1.0301x over previous
import functools

import jax
import jax.numpy as jnp
from jax import lax
from jax.experimental import pallas as pl
from jax.experimental.pallas import tpu as pltpu
from jax.experimental.pallas import tpu_sc as plsc

B, N, V = 32, 16, 100000
K = 64
SEL = 72
NSLOT = SEL + 2
TILES = V // 128
TAIL0 = TILES * 128
GPB = TILES + 1
SLOT_PITCH = 1024
NGV = 128
NPV = 8
A_ROWS = 16

_NEG = -1.0
_BIG = 0x7FFFFFFF


def _groupmax_body(x_ref, p_ref, o_ref):
    nb = A_ROWS // 8
    xs = x_ref[...] * p_ref[...]
    main = jnp.max(xs[:, :TAIL0].reshape(nb, 8, TILES, 128), axis=(1, 3))
    tail = jnp.max(xs[:, TAIL0:].reshape(nb, 8, 32), axis=(1, 2))
    pad = jnp.full((nb, SLOT_PITCH - GPB), _NEG, jnp.float32)
    o_ref[...] = jnp.concatenate(
        [main, tail.reshape(nb, 1), pad], axis=1).reshape(nb, 8, 128)


def _phase_a(raw2d, par2d):
    nb = A_ROWS // 8
    return pl.pallas_call(
        _groupmax_body,
        grid=(B * N // A_ROWS,),
        in_specs=[
            pl.BlockSpec((A_ROWS, V), lambda i: (i, 0)),
            pl.BlockSpec((A_ROWS, 1), lambda i: (i, 0)),
        ],
        out_specs=pl.BlockSpec((nb, 8, 128), lambda i: (i, 0, 0)),
        out_shape=jax.ShapeDtypeStruct(
            (B * N // 8, 8, 128), jnp.float32),
    )(raw2d, par2d)


def _max16(v):
    m = v[0]
    for l in range(1, 16):
        m = jnp.maximum(m, v[l])
    return m


def _argmax16_base(v, base):
    m, mi = v[0], base
    for l in range(1, 16):
        better = v[l] > m
        m = jnp.where(better, v[l], m)
        mi = jnp.where(better, base + l, mi)
    return m, mi


def _argmax16_pair(mv, iv):
    m, mi = mv[0], iv[0]
    for l in range(1, 16):
        better = (mv[l] > m) | ((mv[l] == m) & (iv[l] < mi))
        m = jnp.where(better, mv[l], m)
        mi = jnp.where(better, iv[l], mi)
    return m, mi


def _argmax16_triple(mv, iv, jv):
    m, mi, mj = mv[0], iv[0], jv[0]
    for l in range(1, 16):
        better = (mv[l] > m) | ((mv[l] == m) & (iv[l] < mi))
        m = jnp.where(better, mv[l], m)
        mi = jnp.where(better, iv[l], mi)
        mj = jnp.where(better, jv[l], mj)
    return m, mi, mj


def _select16(vals, sel):
    acc = vals[0]
    for l in range(1, 16):
        acc = jnp.where(sel == l, vals[l], acc)
    return acc


def _rmw_store(ref, lanes, slot, val):
    off = (slot // 16) * 16
    vec = ref[pl.ds(off, 16)]
    ref[pl.ds(off, 16)] = jnp.where(lanes == slot - off, val, vec)


def _phase_b_body(sgm_hbm, par_hbm, raw_hbm,
                  tok_hbm, prb_hbm, pidx_hbm,
                  gm_v, par_v, pvm_v, selgid_v, fb_v, grp_v, tail_v,
                  gbv_v, gbi_v, tok_v, prb_v, pidx_v, sem):
    b = lax.axis_index("s") * 2 + lax.axis_index("c")
    lanes = lax.iota(jnp.int32, 16)

    pltpu.sync_copy(sgm_hbm.at[2 * b], gm_v.at[0])
    pltpu.sync_copy(sgm_hbm.at[2 * b + 1], gm_v.at[1])
    pltpu.sync_copy(par_hbm.at[b], par_v)
    pvec = par_v[pl.ds(0, 16)]
    par_s = [pvec[l] for l in range(16)]

    def vreg_max(i, c):
        ql = i // 64
        r = i - ql * 64
        s = r // 8
        v = r - s * 8
        _rmw_store(pvm_v, lanes, i, _max16(gm_v[ql, s, pl.ds(v * 16, 16)]))
        return c
    lax.fori_loop(0, NGV, vreg_max, 0)

    def pick_group(t, c):
        mv = jnp.full((16,), -2.0, jnp.float32)
        sv = jnp.full((16,), _BIG, jnp.int32)
        for k in range(NPV):
            v = pvm_v[pl.ds(k * 16, 16)]
            take = v > mv
            mv = jnp.where(take, v, mv)
            sv = jnp.where(take, k * 16 + lanes, sv)
        _, kwin = _argmax16_pair(mv, sv)
        ql = kwin // 64
        r = kwin - ql * 64
        s = r // 8
        v = r - s * 8
        vwin = gm_v[ql, s, pl.ds(v * 16, 16)]
        _, slot = _argmax16_base(vwin, kwin * 16)
        _rmw_store(selgid_v, lanes, t, slot)
        vnew = jnp.where(lanes == slot - kwin * 16, jnp.float32(_NEG), vwin)
        gm_v[ql, s, pl.ds(v * 16, 16)] = vnew
        _rmw_store(pvm_v, lanes, kwin, _max16(vnew))
        tt = slot - ql * SLOT_PITCH
        tc = jnp.minimum(tt, TILES - 1)
        pltpu.async_copy(
            raw_hbm.at[pl.ds((2 * b + ql) * 8, 8), pl.ds(tc * 128, 128)],
            grp_v.at[t], sem)
        return c
    lax.fori_loop(0, SEL, pick_group, 0)

    for ql in range(2):
        pltpu.async_copy(
            raw_hbm.at[pl.ds((2 * b + ql) * 8, 8), pl.ds(TAIL0, 32)],
            tail_v.at[ql], sem)

    def drain_tile(j, c):
        pltpu.make_async_copy(raw_hbm.at[pl.ds(0, 8), pl.ds(0, 128)],
                              grp_v.at[j], sem).wait()
        return c
    lax.fori_loop(0, SEL, drain_tile, 0)
    for ql in range(2):
        pltpu.make_async_copy(raw_hbm.at[pl.ds(0, 8), pl.ds(TAIL0, 32)],
                              tail_v.at[ql], sem).wait()

    gbv_v[pl.ds(64, 16)] = jnp.full((16,), -2.0, jnp.float32)
    gbi_v[pl.ds(64, 16)] = jnp.full((16,), _BIG)

    def group_init(j, c):
        gvec = selgid_v[pl.ds((j // 16) * 16, 16)]
        slot = _select16([gvec[l] for l in range(16)], j - (j // 16) * 16)
        ql = slot // SLOT_PITCH
        t = slot - ql * SLOT_PITCH
        dummy = t == TILES
        fb = (ql * 8) * V + t * 128
        bias = jnp.where(dummy, jnp.float32(_NEG), jnp.float32(0.0))
        mv = jnp.full((16,), -2.0, jnp.float32)
        iv = jnp.full((16,), _BIG, jnp.int32)
        zero = jnp.float32(0.0)
        for s in range(8):
            scale = jnp.where(dummy, zero,
                              jnp.where(ql == 0, par_s[s], par_s[8 + s]))
            for v in range(8):
                raw = grp_v[j, s, pl.ds(v * 16, 16)]
                val = raw * scale + bias
                grp_v[j, s, pl.ds(v * 16, 16)] = val
                fids = fb + s * V + v * 16 + lanes
                take = val > mv
                mv = jnp.where(take, val, mv)
                iv = jnp.where(take, fids, iv)
        bv, bi = _argmax16_pair(mv, iv)
        _rmw_store(gbv_v, lanes, j, bv)
        _rmw_store(gbi_v, lanes, j, bi)
        _rmw_store(fb_v, lanes, j, fb)
        return c
    lax.fori_loop(0, SEL, group_init, 0)

    neg16 = jnp.full((16,), _NEG, jnp.float32)
    for ql in range(2):
        fb = (ql * 8) * V + TAIL0
        mv = jnp.full((16,), -2.0, jnp.float32)
        iv = jnp.full((16,), _BIG, jnp.int32)
        for s in range(8):
            scale = par_s[ql * 8 + s]
            for v in range(8):
                if v < 2:
                    val = tail_v[ql, s, pl.ds(v * 16, 16)] * scale
                else:
                    val = neg16
                grp_v[SEL + ql, s, pl.ds(v * 16, 16)] = val
                fids = fb + s * V + v * 16 + lanes
                take = val > mv
                mv = jnp.where(take, val, mv)
                iv = jnp.where(take, fids, iv)
        bv, bi = _argmax16_pair(mv, iv)
        _rmw_store(gbv_v, lanes, SEL + ql, bv)
        _rmw_store(gbi_v, lanes, SEL + ql, bi)
        _rmw_store(fb_v, lanes, SEL + ql, fb)

    def round_t(t, c):
        mv = jnp.full((16,), -2.0, jnp.float32)
        iv = jnp.full((16,), _BIG, jnp.int32)
        jv = jnp.full((16,), _BIG, jnp.int32)
        for k in range(5):
            v = gbv_v[pl.ds(k * 16, 16)]
            fi = gbi_v[pl.ds(k * 16, 16)]
            take = (v > mv) | ((v == mv) & (fi < iv))
            mv = jnp.where(take, v, mv)
            iv = jnp.where(take, fi, iv)
            jv = jnp.where(take, k * 16 + lanes, jv)
        m, fwin, jwin = _argmax16_triple(mv, iv, jv)

        _rmw_store(prb_v, lanes, t, m)
        _rmw_store(tok_v, lanes, t, fwin % V)
        _rmw_store(pidx_v, lanes, t, fwin // V)

        n = fwin // V
        col = fwin - n * V
        s_r = n - (n // 8) * 8
        fvec = fb_v[pl.ds((jwin // 16) * 16, 16)]
        fb = _select16([fvec[l] for l in range(16)], jwin - (jwin // 16) * 16)
        loc = col - (fb - (fb // V) * V)
        vr = loc // 16
        lpos = loc - vr * 16

        vec = grp_v[jwin, s_r, pl.ds(vr * 16, 16)]
        grp_v[jwin, s_r, pl.ds(vr * 16, 16)] = jnp.where(
            lanes == lpos, jnp.float32(_NEG), vec)

        mv1 = jnp.full((16,), -2.0, jnp.float32)
        iv1 = jnp.full((16,), _BIG, jnp.int32)
        for s in range(8):
            for v in range(8):
                val = grp_v[jwin, s, pl.ds(v * 16, 16)]
                fids = fb + s * V + v * 16 + lanes
                take = val > mv1
                mv1 = jnp.where(take, val, mv1)
                iv1 = jnp.where(take, fids, iv1)
        bv, bi = _argmax16_pair(mv1, iv1)
        _rmw_store(gbv_v, lanes, jwin, bv)
        _rmw_store(gbi_v, lanes, jwin, bi)
        return c
    lax.fori_loop(0, K, round_t, 0)

    pltpu.sync_copy(tok_v, tok_hbm.at[b])
    pltpu.sync_copy(prb_v, prb_hbm.at[b])
    pltpu.sync_copy(pidx_v, pidx_hbm.at[b])


def _phase_b(sgm, parent_probs, raw2d):
    mesh = plsc.VectorSubcoreMesh(core_axis_name="c", subcore_axis_name="s")
    fn = functools.partial(
        pl.kernel,
        mesh=mesh,
        out_type=[
            jax.ShapeDtypeStruct((B, K), jnp.int32),
            jax.ShapeDtypeStruct((B, K), jnp.float32),
            jax.ShapeDtypeStruct((B, K), jnp.int32),
        ],
        scratch_types=[
            pltpu.VMEM((2, 8, 128), jnp.float32),
            pltpu.VMEM((16,), jnp.float32),
            pltpu.VMEM((NPV * 16,), jnp.float32),
            pltpu.VMEM((80,), jnp.int32),
            pltpu.VMEM((80,), jnp.int32),
            pltpu.VMEM((NSLOT, 8, 128), jnp.float32),
            pltpu.VMEM((2, 8, 32), jnp.float32),
            pltpu.VMEM((80,), jnp.float32),
            pltpu.VMEM((80,), jnp.int32),
            pltpu.VMEM((K,), jnp.int32),
            pltpu.VMEM((K,), jnp.float32),
            pltpu.VMEM((K,), jnp.int32),
            pltpu.SemaphoreType.DMA,
        ],
    )(_phase_b_body)
    return fn(sgm, parent_probs, raw2d)


def kernel(sampled_probs, parent_probs, sample_k, sample_min_prob):
    del sample_k, sample_min_prob
    raw2d = sampled_probs.reshape(B * N, V)
    par2d = parent_probs.reshape(B * N, 1)
    sgm = _phase_a(raw2d, par2d)
    tok, prb, pidx = _phase_b(sgm, parent_probs, raw2d)
    return tok, prb, pidx


# speedup vs baseline: 90.6856x; 1.0582x over previous
import functools

import jax
import jax.numpy as jnp
from jax import lax
from jax.experimental import pallas as pl
from jax.experimental.pallas import tpu as pltpu
from jax.experimental.pallas import tpu_sc as plsc

B, N, V = 32, 16, 100000
K = 64
SEL = 72
NSLOT = SEL + 2
TILES = V // 128
TAIL0 = TILES * 128
GPB = TILES + 1
SLOT_PITCH = 1024
NGV = 128
NPV = 8
A_ROWS = 32

_NEG = -1.0
_BIG = 0x7FFFFFFF


def _groupmax_body(x_ref, p_ref, o_ref):
    nb = A_ROWS // 8
    xs = x_ref[...] * p_ref[...]
    main = jnp.max(xs[:, :TAIL0].reshape(nb, 8, TILES, 128), axis=(1, 3))
    tail = jnp.max(xs[:, TAIL0:].reshape(nb, 8, 32), axis=(1, 2))
    pad = jnp.full((nb, SLOT_PITCH - GPB), _NEG, jnp.float32)
    o_ref[...] = jnp.concatenate(
        [main, tail.reshape(nb, 1), pad], axis=1).reshape(nb, 8, 128)


def _phase_a(raw2d, par2d):
    nb = A_ROWS // 8
    return pl.pallas_call(
        _groupmax_body,
        grid=(B * N // A_ROWS,),
        in_specs=[
            pl.BlockSpec((A_ROWS, V), lambda i: (i, 0)),
            pl.BlockSpec((A_ROWS, 1), lambda i: (i, 0)),
        ],
        out_specs=pl.BlockSpec((nb, 8, 128), lambda i: (i, 0, 0)),
        out_shape=jax.ShapeDtypeStruct(
            (B * N // 8, 8, 128), jnp.float32),
    )(raw2d, par2d)


def _max16(v):
    m = v[0]
    for l in range(1, 16):
        m = jnp.maximum(m, v[l])
    return m


def _argmax16_base(v, base):
    m, mi = v[0], base
    for l in range(1, 16):
        better = v[l] > m
        m = jnp.where(better, v[l], m)
        mi = jnp.where(better, base + l, mi)
    return m, mi


def _argmax16_pair(mv, iv):
    m, mi = mv[0], iv[0]
    for l in range(1, 16):
        better = (mv[l] > m) | ((mv[l] == m) & (iv[l] < mi))
        m = jnp.where(better, mv[l], m)
        mi = jnp.where(better, iv[l], mi)
    return m, mi


def _argmax16_triple(mv, iv, jv):
    m, mi, mj = mv[0], iv[0], jv[0]
    for l in range(1, 16):
        better = (mv[l] > m) | ((mv[l] == m) & (iv[l] < mi))
        m = jnp.where(better, mv[l], m)
        mi = jnp.where(better, iv[l], mi)
        mj = jnp.where(better, jv[l], mj)
    return m, mi, mj


def _select16(vals, sel):
    acc = vals[0]
    for l in range(1, 16):
        acc = jnp.where(sel == l, vals[l], acc)
    return acc


def _rmw_store(ref, lanes, slot, val):
    off = (slot // 16) * 16
    vec = ref[pl.ds(off, 16)]
    ref[pl.ds(off, 16)] = jnp.where(lanes == slot - off, val, vec)


def _phase_b_body(sgm_hbm, par_hbm, raw_hbm,
                  tok_hbm, prb_hbm, pidx_hbm,
                  gm_v, par_v, pvm_v, selgid_v, fb_v, grp_v, tail_v,
                  gbv_v, gbi_v, tok_v, prb_v, pidx_v, sem):
    b = lax.axis_index("s") * 2 + lax.axis_index("c")
    lanes = lax.iota(jnp.int32, 16)

    pltpu.sync_copy(sgm_hbm.at[2 * b], gm_v.at[0])
    pltpu.sync_copy(sgm_hbm.at[2 * b + 1], gm_v.at[1])
    pltpu.sync_copy(par_hbm.at[b], par_v)
    pvec = par_v[pl.ds(0, 16)]
    par_s = [pvec[l] for l in range(16)]

    def vreg_max(i, c):
        ql = i // 64
        r = i - ql * 64
        s = r // 8
        v = r - s * 8
        _rmw_store(pvm_v, lanes, i, _max16(gm_v[ql, s, pl.ds(v * 16, 16)]))
        return c
    lax.fori_loop(0, NGV, vreg_max, 0)

    def pick_group(t, c):
        mv = jnp.full((16,), -2.0, jnp.float32)
        sv = jnp.full((16,), _BIG, jnp.int32)
        for k in range(NPV):
            v = pvm_v[pl.ds(k * 16, 16)]
            take = v > mv
            mv = jnp.where(take, v, mv)
            sv = jnp.where(take, k * 16 + lanes, sv)
        _, kwin = _argmax16_pair(mv, sv)
        ql = kwin // 64
        r = kwin - ql * 64
        s = r // 8
        v = r - s * 8
        vwin = gm_v[ql, s, pl.ds(v * 16, 16)]
        _, slot = _argmax16_base(vwin, kwin * 16)
        _rmw_store(selgid_v, lanes, t, slot)
        vnew = jnp.where(lanes == slot - kwin * 16, jnp.float32(_NEG), vwin)
        gm_v[ql, s, pl.ds(v * 16, 16)] = vnew
        _rmw_store(pvm_v, lanes, kwin, _max16(vnew))
        tt = slot - ql * SLOT_PITCH
        tc = jnp.minimum(tt, TILES - 1)
        pltpu.async_copy(
            raw_hbm.at[pl.ds((2 * b + ql) * 8, 8), pl.ds(tc * 128, 128)],
            grp_v.at[t], sem)
        return c
    lax.fori_loop(0, SEL, pick_group, 0)

    for ql in range(2):
        pltpu.async_copy(
            raw_hbm.at[pl.ds((2 * b + ql) * 8, 8), pl.ds(TAIL0, 32)],
            tail_v.at[ql], sem)

    def drain_tile(j, c):
        pltpu.make_async_copy(raw_hbm.at[pl.ds(0, 8), pl.ds(0, 128)],
                              grp_v.at[j], sem).wait()
        return c
    lax.fori_loop(0, SEL, drain_tile, 0)
    for ql in range(2):
        pltpu.make_async_copy(raw_hbm.at[pl.ds(0, 8), pl.ds(TAIL0, 32)],
                              tail_v.at[ql], sem).wait()

    gbv_v[pl.ds(64, 16)] = jnp.full((16,), -2.0, jnp.float32)
    gbi_v[pl.ds(64, 16)] = jnp.full((16,), _BIG)

    def group_init(j, c):
        gvec = selgid_v[pl.ds((j // 16) * 16, 16)]
        slot = _select16([gvec[l] for l in range(16)], j - (j // 16) * 16)
        ql = slot // SLOT_PITCH
        t = slot - ql * SLOT_PITCH
        dummy = t == TILES
        fb = (ql * 8) * V + t * 128
        bias = jnp.where(dummy, jnp.float32(_NEG), jnp.float32(0.0))
        mv = jnp.full((16,), -2.0, jnp.float32)
        iv = jnp.full((16,), _BIG, jnp.int32)
        zero = jnp.float32(0.0)
        for s in range(8):
            scale = jnp.where(dummy, zero,
                              jnp.where(ql == 0, par_s[s], par_s[8 + s]))
            for v in range(8):
                raw = grp_v[j, s, pl.ds(v * 16, 16)]
                val = raw * scale + bias
                grp_v[j, s, pl.ds(v * 16, 16)] = val
                fids = fb + s * V + v * 16 + lanes
                take = val > mv
                mv = jnp.where(take, val, mv)
                iv = jnp.where(take, fids, iv)
        bv, bi = _argmax16_pair(mv, iv)
        _rmw_store(gbv_v, lanes, j, bv)
        _rmw_store(gbi_v, lanes, j, bi)
        _rmw_store(fb_v, lanes, j, fb)
        return c
    lax.fori_loop(0, SEL, group_init, 0)

    neg16 = jnp.full((16,), _NEG, jnp.float32)
    for ql in range(2):
        fb = (ql * 8) * V + TAIL0
        mv = jnp.full((16,), -2.0, jnp.float32)
        iv = jnp.full((16,), _BIG, jnp.int32)
        for s in range(8):
            scale = par_s[ql * 8 + s]
            for v in range(8):
                if v < 2:
                    val = tail_v[ql, s, pl.ds(v * 16, 16)] * scale
                else:
                    val = neg16
                grp_v[SEL + ql, s, pl.ds(v * 16, 16)] = val
                fids = fb + s * V + v * 16 + lanes
                take = val > mv
                mv = jnp.where(take, val, mv)
                iv = jnp.where(take, fids, iv)
        bv, bi = _argmax16_pair(mv, iv)
        _rmw_store(gbv_v, lanes, SEL + ql, bv)
        _rmw_store(gbi_v, lanes, SEL + ql, bi)
        _rmw_store(fb_v, lanes, SEL + ql, fb)

    def round_t(t, c):
        mv = jnp.full((16,), -2.0, jnp.float32)
        iv = jnp.full((16,), _BIG, jnp.int32)
        jv = jnp.full((16,), _BIG, jnp.int32)
        for k in range(5):
            v = gbv_v[pl.ds(k * 16, 16)]
            fi = gbi_v[pl.ds(k * 16, 16)]
            take = (v > mv) | ((v == mv) & (fi < iv))
            mv = jnp.where(take, v, mv)
            iv = jnp.where(take, fi, iv)
            jv = jnp.where(take, k * 16 + lanes, jv)
        m, fwin, jwin = _argmax16_triple(mv, iv, jv)

        _rmw_store(prb_v, lanes, t, m)
        _rmw_store(tok_v, lanes, t, fwin % V)
        _rmw_store(pidx_v, lanes, t, fwin // V)

        n = fwin // V
        col = fwin - n * V
        s_r = n - (n // 8) * 8
        fvec = fb_v[pl.ds((jwin // 16) * 16, 16)]
        fb = _select16([fvec[l] for l in range(16)], jwin - (jwin // 16) * 16)
        loc = col - (fb - (fb // V) * V)
        vr = loc // 16
        lpos = loc - vr * 16

        vec = grp_v[jwin, s_r, pl.ds(vr * 16, 16)]
        grp_v[jwin, s_r, pl.ds(vr * 16, 16)] = jnp.where(
            lanes == lpos, jnp.float32(_NEG), vec)

        mv1 = jnp.full((16,), -2.0, jnp.float32)
        iv1 = jnp.full((16,), _BIG, jnp.int32)
        for s in range(8):
            for v in range(8):
                val = grp_v[jwin, s, pl.ds(v * 16, 16)]
                fids = fb + s * V + v * 16 + lanes
                take = val > mv1
                mv1 = jnp.where(take, val, mv1)
                iv1 = jnp.where(take, fids, iv1)
        bv, bi = _argmax16_pair(mv1, iv1)
        _rmw_store(gbv_v, lanes, jwin, bv)
        _rmw_store(gbi_v, lanes, jwin, bi)
        return c
    lax.fori_loop(0, K, round_t, 0)

    pltpu.sync_copy(tok_v, tok_hbm.at[b])
    pltpu.sync_copy(prb_v, prb_hbm.at[b])
    pltpu.sync_copy(pidx_v, pidx_hbm.at[b])


def _phase_b(sgm, parent_probs, raw2d):
    mesh = plsc.VectorSubcoreMesh(core_axis_name="c", subcore_axis_name="s")
    fn = functools.partial(
        pl.kernel,
        mesh=mesh,
        out_type=[
            jax.ShapeDtypeStruct((B, K), jnp.int32),
            jax.ShapeDtypeStruct((B, K), jnp.float32),
            jax.ShapeDtypeStruct((B, K), jnp.int32),
        ],
        scratch_types=[
            pltpu.VMEM((2, 8, 128), jnp.float32),
            pltpu.VMEM((16,), jnp.float32),
            pltpu.VMEM((NPV * 16,), jnp.float32),
            pltpu.VMEM((80,), jnp.int32),
            pltpu.VMEM((80,), jnp.int32),
            pltpu.VMEM((NSLOT, 8, 128), jnp.float32),
            pltpu.VMEM((2, 8, 32), jnp.float32),
            pltpu.VMEM((80,), jnp.float32),
            pltpu.VMEM((80,), jnp.int32),
            pltpu.VMEM((K,), jnp.int32),
            pltpu.VMEM((K,), jnp.float32),
            pltpu.VMEM((K,), jnp.int32),
            pltpu.SemaphoreType.DMA,
        ],
    )(_phase_b_body)
    return fn(sgm, parent_probs, raw2d)


def kernel(sampled_probs, parent_probs, sample_k, sample_min_prob):
    del sample_k, sample_min_prob
    raw2d = sampled_probs.reshape(B * N, V)
    par2d = parent_probs.reshape(B * N, 1)
    sgm = _phase_a(raw2d, par2d)
    tok, prb, pidx = _phase_b(sgm, parent_probs, raw2d)
    return tok, prb, pidx


# speedup vs baseline: 90.9731x; 1.0032x over previous
import functools

import jax
import jax.numpy as jnp
from jax import lax
from jax.experimental import pallas as pl
from jax.experimental.pallas import tpu as pltpu
from jax.experimental.pallas import tpu_sc as plsc

B, N, V = 32, 16, 100000
K = 64
SEL = 72
NSLOT = SEL + 2
TILES = V // 128
TAIL0 = TILES * 128
GPB = TILES + 1
SLOT_PITCH = 1024
NGV = 128
NPV = 8
A_ROWS = 64

_NEG = -1.0
_BIG = 0x7FFFFFFF


def _groupmax_body(x_ref, p_ref, o_ref):
    nb = A_ROWS // 8
    xs = x_ref[...] * p_ref[...]
    main = jnp.max(xs[:, :TAIL0].reshape(nb, 8, TILES, 128), axis=(1, 3))
    tail = jnp.max(xs[:, TAIL0:].reshape(nb, 8, 32), axis=(1, 2))
    pad = jnp.full((nb, SLOT_PITCH - GPB), _NEG, jnp.float32)
    o_ref[...] = jnp.concatenate(
        [main, tail.reshape(nb, 1), pad], axis=1).reshape(nb, 8, 128)


def _phase_a(raw2d, par2d):
    nb = A_ROWS // 8
    return pl.pallas_call(
        _groupmax_body,
        grid=(B * N // A_ROWS,),
        in_specs=[
            pl.BlockSpec((A_ROWS, V), lambda i: (i, 0)),
            pl.BlockSpec((A_ROWS, 1), lambda i: (i, 0)),
        ],
        out_specs=pl.BlockSpec((nb, 8, 128), lambda i: (i, 0, 0)),
        out_shape=jax.ShapeDtypeStruct(
            (B * N // 8, 8, 128), jnp.float32),
    )(raw2d, par2d)


def _max16(v):
    m = v[0]
    for l in range(1, 16):
        m = jnp.maximum(m, v[l])
    return m


def _argmax16_base(v, base):
    m, mi = v[0], base
    for l in range(1, 16):
        better = v[l] > m
        m = jnp.where(better, v[l], m)
        mi = jnp.where(better, base + l, mi)
    return m, mi


def _argmax16_pair(mv, iv):
    m, mi = mv[0], iv[0]
    for l in range(1, 16):
        better = (mv[l] > m) | ((mv[l] == m) & (iv[l] < mi))
        m = jnp.where(better, mv[l], m)
        mi = jnp.where(better, iv[l], mi)
    return m, mi


def _argmax16_triple(mv, iv, jv):
    m, mi, mj = mv[0], iv[0], jv[0]
    for l in range(1, 16):
        better = (mv[l] > m) | ((mv[l] == m) & (iv[l] < mi))
        m = jnp.where(better, mv[l], m)
        mi = jnp.where(better, iv[l], mi)
        mj = jnp.where(better, jv[l], mj)
    return m, mi, mj


def _select16(vals, sel):
    acc = vals[0]
    for l in range(1, 16):
        acc = jnp.where(sel == l, vals[l], acc)
    return acc


def _rmw_store(ref, lanes, slot, val):
    off = (slot // 16) * 16
    vec = ref[pl.ds(off, 16)]
    ref[pl.ds(off, 16)] = jnp.where(lanes == slot - off, val, vec)


def _phase_b_body(sgm_hbm, par_hbm, raw_hbm,
                  tok_hbm, prb_hbm, pidx_hbm,
                  gm_v, par_v, pvm_v, selgid_v, fb_v, grp_v, tail_v,
                  gbv_v, gbi_v, tok_v, prb_v, pidx_v, sem):
    b = lax.axis_index("s") * 2 + lax.axis_index("c")
    lanes = lax.iota(jnp.int32, 16)

    pltpu.sync_copy(sgm_hbm.at[2 * b], gm_v.at[0])
    pltpu.sync_copy(sgm_hbm.at[2 * b + 1], gm_v.at[1])
    pltpu.sync_copy(par_hbm.at[b], par_v)
    pvec = par_v[pl.ds(0, 16)]
    par_s = [pvec[l] for l in range(16)]

    def vreg_max(i, c):
        ql = i // 64
        r = i - ql * 64
        s = r // 8
        v = r - s * 8
        _rmw_store(pvm_v, lanes, i, _max16(gm_v[ql, s, pl.ds(v * 16, 16)]))
        return c
    lax.fori_loop(0, NGV, vreg_max, 0)

    def pick_group(t, c):
        mv = jnp.full((16,), -2.0, jnp.float32)
        sv = jnp.full((16,), _BIG, jnp.int32)
        for k in range(NPV):
            v = pvm_v[pl.ds(k * 16, 16)]
            take = v > mv
            mv = jnp.where(take, v, mv)
            sv = jnp.where(take, k * 16 + lanes, sv)
        _, kwin = _argmax16_pair(mv, sv)
        ql = kwin // 64
        r = kwin - ql * 64
        s = r // 8
        v = r - s * 8
        vwin = gm_v[ql, s, pl.ds(v * 16, 16)]
        _, slot = _argmax16_base(vwin, kwin * 16)
        _rmw_store(selgid_v, lanes, t, slot)
        vnew = jnp.where(lanes == slot - kwin * 16, jnp.float32(_NEG), vwin)
        gm_v[ql, s, pl.ds(v * 16, 16)] = vnew
        _rmw_store(pvm_v, lanes, kwin, _max16(vnew))
        tt = slot - ql * SLOT_PITCH
        tc = jnp.minimum(tt, TILES - 1)
        pltpu.async_copy(
            raw_hbm.at[pl.ds((2 * b + ql) * 8, 8), pl.ds(tc * 128, 128)],
            grp_v.at[t], sem)
        return c
    lax.fori_loop(0, SEL, pick_group, 0)

    for ql in range(2):
        pltpu.async_copy(
            raw_hbm.at[pl.ds((2 * b + ql) * 8, 8), pl.ds(TAIL0, 32)],
            tail_v.at[ql], sem)

    def drain_tile(j, c):
        pltpu.make_async_copy(raw_hbm.at[pl.ds(0, 8), pl.ds(0, 128)],
                              grp_v.at[j], sem).wait()
        return c
    lax.fori_loop(0, SEL, drain_tile, 0)
    for ql in range(2):
        pltpu.make_async_copy(raw_hbm.at[pl.ds(0, 8), pl.ds(TAIL0, 32)],
                              tail_v.at[ql], sem).wait()

    gbv_v[pl.ds(64, 16)] = jnp.full((16,), -2.0, jnp.float32)
    gbi_v[pl.ds(64, 16)] = jnp.full((16,), _BIG)

    def group_init(j, c):
        gvec = selgid_v[pl.ds((j // 16) * 16, 16)]
        slot = _select16([gvec[l] for l in range(16)], j - (j // 16) * 16)
        ql = slot // SLOT_PITCH
        t = slot - ql * SLOT_PITCH
        dummy = t == TILES
        fb = (ql * 8) * V + t * 128
        bias = jnp.where(dummy, jnp.float32(_NEG), jnp.float32(0.0))
        mv = jnp.full((16,), -2.0, jnp.float32)
        iv = jnp.full((16,), _BIG, jnp.int32)
        zero = jnp.float32(0.0)
        for s in range(8):
            scale = jnp.where(dummy, zero,
                              jnp.where(ql == 0, par_s[s], par_s[8 + s]))
            for v in range(8):
                raw = grp_v[j, s, pl.ds(v * 16, 16)]
                val = raw * scale + bias
                grp_v[j, s, pl.ds(v * 16, 16)] = val
                fids = fb + s * V + v * 16 + lanes
                take = val > mv
                mv = jnp.where(take, val, mv)
                iv = jnp.where(take, fids, iv)
        bv, bi = _argmax16_pair(mv, iv)
        _rmw_store(gbv_v, lanes, j, bv)
        _rmw_store(gbi_v, lanes, j, bi)
        _rmw_store(fb_v, lanes, j, fb)
        return c
    lax.fori_loop(0, SEL, group_init, 0)

    neg16 = jnp.full((16,), _NEG, jnp.float32)
    for ql in range(2):
        fb = (ql * 8) * V + TAIL0
        mv = jnp.full((16,), -2.0, jnp.float32)
        iv = jnp.full((16,), _BIG, jnp.int32)
        for s in range(8):
            scale = par_s[ql * 8 + s]
            for v in range(8):
                if v < 2:
                    val = tail_v[ql, s, pl.ds(v * 16, 16)] * scale
                else:
                    val = neg16
                grp_v[SEL + ql, s, pl.ds(v * 16, 16)] = val
                fids = fb + s * V + v * 16 + lanes
                take = val > mv
                mv = jnp.where(take, val, mv)
                iv = jnp.where(take, fids, iv)
        bv, bi = _argmax16_pair(mv, iv)
        _rmw_store(gbv_v, lanes, SEL + ql, bv)
        _rmw_store(gbi_v, lanes, SEL + ql, bi)
        _rmw_store(fb_v, lanes, SEL + ql, fb)

    def round_t(t, c):
        mv = jnp.full((16,), -2.0, jnp.float32)
        iv = jnp.full((16,), _BIG, jnp.int32)
        jv = jnp.full((16,), _BIG, jnp.int32)
        for k in range(5):
            v = gbv_v[pl.ds(k * 16, 16)]
            fi = gbi_v[pl.ds(k * 16, 16)]
            take = (v > mv) | ((v == mv) & (fi < iv))
            mv = jnp.where(take, v, mv)
            iv = jnp.where(take, fi, iv)
            jv = jnp.where(take, k * 16 + lanes, jv)
        m, fwin, jwin = _argmax16_triple(mv, iv, jv)

        _rmw_store(prb_v, lanes, t, m)
        _rmw_store(tok_v, lanes, t, fwin % V)
        _rmw_store(pidx_v, lanes, t, fwin // V)

        n = fwin // V
        col = fwin - n * V
        s_r = n - (n // 8) * 8
        fvec = fb_v[pl.ds((jwin // 16) * 16, 16)]
        fb = _select16([fvec[l] for l in range(16)], jwin - (jwin // 16) * 16)
        loc = col - (fb - (fb // V) * V)
        vr = loc // 16
        lpos = loc - vr * 16

        vec = grp_v[jwin, s_r, pl.ds(vr * 16, 16)]
        grp_v[jwin, s_r, pl.ds(vr * 16, 16)] = jnp.where(
            lanes == lpos, jnp.float32(_NEG), vec)

        mv1 = jnp.full((16,), -2.0, jnp.float32)
        iv1 = jnp.full((16,), _BIG, jnp.int32)
        for s in range(8):
            for v in range(8):
                val = grp_v[jwin, s, pl.ds(v * 16, 16)]
                fids = fb + s * V + v * 16 + lanes
                take = val > mv1
                mv1 = jnp.where(take, val, mv1)
                iv1 = jnp.where(take, fids, iv1)
        bv, bi = _argmax16_pair(mv1, iv1)
        _rmw_store(gbv_v, lanes, jwin, bv)
        _rmw_store(gbi_v, lanes, jwin, bi)
        return c
    lax.fori_loop(0, K, round_t, 0)

    pltpu.sync_copy(tok_v, tok_hbm.at[b])
    pltpu.sync_copy(prb_v, prb_hbm.at[b])
    pltpu.sync_copy(pidx_v, pidx_hbm.at[b])


def _phase_b(sgm, parent_probs, raw2d):
    mesh = plsc.VectorSubcoreMesh(core_axis_name="c", subcore_axis_name="s")
    fn = functools.partial(
        pl.kernel,
        mesh=mesh,
        out_type=[
            jax.ShapeDtypeStruct((B, K), jnp.int32),
            jax.ShapeDtypeStruct((B, K), jnp.float32),
            jax.ShapeDtypeStruct((B, K), jnp.int32),
        ],
        scratch_types=[
            pltpu.VMEM((2, 8, 128), jnp.float32),
            pltpu.VMEM((16,), jnp.float32),
            pltpu.VMEM((NPV * 16,), jnp.float32),
            pltpu.VMEM((80,), jnp.int32),
            pltpu.VMEM((80,), jnp.int32),
            pltpu.VMEM((NSLOT, 8, 128), jnp.float32),
            pltpu.VMEM((2, 8, 32), jnp.float32),
            pltpu.VMEM((80,), jnp.float32),
            pltpu.VMEM((80,), jnp.int32),
            pltpu.VMEM((K,), jnp.int32),
            pltpu.VMEM((K,), jnp.float32),
            pltpu.VMEM((K,), jnp.int32),
            pltpu.SemaphoreType.DMA,
        ],
    )(_phase_b_body)
    return fn(sgm, parent_probs, raw2d)


def kernel(sampled_probs, parent_probs, sample_k, sample_min_prob):
    del sample_k, sample_min_prob
    raw2d = sampled_probs.reshape(B * N, V)
    par2d = parent_probs.reshape(B * N, 1)
    sgm = _phase_a(raw2d, par2d)
    tok, prb, pidx = _phase_b(sgm, parent_probs, raw2d)
    return tok, prb, pidx
